# Initial kernel scaffold; baseline (speedup 1.0000x reference)
import jax
import jax.numpy as jnp
from jax.experimental import pallas as pl


def kernel(query, key_in, value, WQ, WK, WV):
    raise NotImplementedError("write your pallas kernel here")


# speedup vs baseline: 4.2114x; 4.2114x over previous
import functools

import jax
import jax.numpy as jnp
from jax import lax
from jax.experimental import pallas as pl
from jax.experimental.pallas import tpu as pltpu
from jax.experimental.pallas import tpu_sc as plsc

F = 2048
HF = 1024
QK = 64
LQ = 2048
LK = 8192
TOPK = 32
GATE = 0.1
TEMP = QK ** -0.5
EPS = 1e-12

NW = 32
RPW = LQ // NW
NBINS = 512
CAP = 1024


def _l2n(x):
    n = jnp.sqrt(jnp.sum(x * x, axis=1, keepdims=True))
    return x / jnp.maximum(n, EPS)


def _wk_body(kin_ref, wk_ref, o_ref):
    y = lax.dot_general(kin_ref[...], wk_ref[...], (((1,), (1,)), ((), ())),
                        preferred_element_type=jnp.float32)
    o_ref[...] = _l2n(y)


def _scores_body(q_ref, wq_ref, wk_ref, s_ref, m_ref, d_ref):
    wq = lax.dot_general(q_ref[...], wq_ref[...], (((1,), (1,)), ((), ())),
                         preferred_element_type=jnp.float32)
    wq = _l2n(wq)
    s = lax.dot_general(wq, wk_ref[...], (((1,), (1,)), ((), ())),
                        preferred_element_type=jnp.float32)
    s_ref[...] = s
    m = jnp.max(s, axis=1)
    m_ref[...] = m
    d_ref[...] = jnp.sum(jnp.exp((s - m[:, None]) * TEMP), axis=1)


def _wv_body(v_ref, wv_ref, o_ref):
    y = lax.dot_general(v_ref[...], wv_ref[...], (((1,), (1,)), ((), ())),
                        preferred_element_type=jnp.float32)
    o_ref[...] = _l2n(y)


def _final_body(p_ref, s_ref, m_ref, q_ref, o_ref):
    g = lax.dot_general(p_ref[...], s_ref[...], (((1,), (1,)), ((), ())),
                        preferred_element_type=jnp.float32)
    msk = (m_ref[...] > GATE).astype(jnp.float32)
    o_ref[...] = g * msk[:, None] + q_ref[...]


def _merge_keep_top(ak, ai, bk, bi):
    rbk = lax.rev(bk, (0,))
    rbi = lax.rev(bi, (0,))
    m = ak >= rbk
    hk = jnp.where(m, ak, rbk)
    hi = jnp.where(m, ai, rbi)
    return plsc.sort_key_val(hk, hi)


def _merge32(ak, ai, bk, bi):
    rbk = lax.rev(bk, (0,))
    rbi = lax.rev(bi, (0,))
    m = ak <= rbk
    lk_ = jnp.where(m, ak, rbk)
    li_ = jnp.where(m, ai, rbi)
    hk_ = jnp.where(m, rbk, ak)
    hi_ = jnp.where(m, rbi, ai)
    lk2, li2 = plsc.sort_key_val(lk_, li_)
    hk2, hi2 = plsc.sort_key_val(hk_, hi_)
    return lk2, li2, hk2, hi2


def _sc_topk_body(scores_hbm, rmax_hbm, den_hbm, wv_hbm, p_hbm, s_hbm,
                  rowbuf, hist, cval, cidx, rm_loc, dn_loc, p_loc, s_loc,
                  gi_loc, gsem):
    wid = lax.axis_index("s") * 2 + lax.axis_index("c")
    base = wid * RPW
    pltpu.sync_copy(rmax_hbm.at[pl.ds(base, RPW)], rm_loc)
    pltpu.sync_copy(den_hbm.at[pl.ds(base, RPW)], dn_loc)
    iota16 = lax.iota(jnp.int32, 16)
    ones16 = jnp.ones((16,), jnp.int32)

    def row_step(r, carry_unused):
        pltpu.sync_copy(scores_hbm.at[pl.ds((base + r) * LK, LK)], rowbuf)
        for i in range(NBINS // 16):
            hist[pl.ds(i * 16, 16)] = jnp.zeros((16,), jnp.int32)

        def h_step(i, _):
            v = rowbuf[pl.ds(i * 16, 16)]
            b = jnp.clip(((v + 1.0) * (NBINS / 2.0)).astype(jnp.int32),
                         0, NBINS - 1)
            plsc.addupdate_scatter(hist, [b], ones16)
            return 0

        lax.fori_loop(0, LK // 16, h_step, 0, unroll=4)

        def t_step(i, carry):
            cum, bfound, bbin = carry
            blk_id = (NBINS // 16 - 1) - i
            blk = hist[pl.ds(blk_id * 16, 16)]
            rcum = plsc.cumsum(lax.rev(blk, (0,))) + cum
            m = rcum >= TOPK
            any_m = jnp.max(m.astype(jnp.int32))
            ffs = jnp.max(plsc.all_reduce_ffs(m))
            cand_bin = blk_id * 16 + 15 - ffs
            bbin = jnp.where(bfound == 0,
                             jnp.where(any_m == 1, cand_bin, bbin), bbin)
            bfound = jnp.maximum(bfound, any_m)
            return (cum + jnp.sum(blk), bfound, bbin)

        _, _, bbin = lax.fori_loop(0, NBINS // 16, t_step,
                                   (jnp.int32(0), jnp.int32(0), jnp.int32(0)))
        thresh = bbin.astype(jnp.float32) * (2.0 / NBINS) - 1.0

        def c_step(i, ptr):
            v = rowbuf[pl.ds(i * 16, 16)]
            m = v >= thresh
            cnt = jnp.max(plsc.all_reduce_population_count(m))
            plsc.store_compressed(cval.at[pl.ds(ptr, 16)], v, mask=m)
            plsc.store_compressed(cidx.at[pl.ds(ptr, 16)], i * 16 + iota16,
                                  mask=m)
            return jnp.minimum(ptr + cnt, CAP - 16)

        ptr = lax.fori_loop(0, LK // 16, c_step, jnp.int32(0), unroll=4)
        cval[pl.ds(ptr, 16)] = jnp.full((16,), -2.0, jnp.float32)
        cidx[pl.ds(ptr, 16)] = jnp.zeros((16,), jnp.int32)

        def s_step(t, st):
            lok, loi, hik, hii = st
            ck = cval[pl.ds(t * 16, 16)]
            ci = cidx[pl.ds(t * 16, 16)]
            nk, ni = plsc.sort_key_val(ck, ci)
            h1k, h1i = _merge_keep_top(lok, loi, nk, ni)
            return _merge32(h1k, h1i, hik, hii)

        init = (jnp.full((16,), -3.0, jnp.float32), jnp.zeros((16,), jnp.int32),
                jnp.full((16,), -3.0, jnp.float32), jnp.zeros((16,), jnp.int32))
        lok, loi, hik, hii = lax.fori_loop(0, (ptr + 15) // 16, s_step, init)

        kd0 = lax.rev(hik, (0,))
        kd1 = lax.rev(lok, (0,))
        id0 = lax.rev(hii, (0,))
        id1 = lax.rev(loi, (0,))
        rmv = plsc.load_gather(rm_loc, [jnp.broadcast_to(r, (16,)).astype(jnp.int32)])
        dnv = plsc.load_gather(dn_loc, [jnp.broadcast_to(r, (16,)).astype(jnp.int32)])
        p_loc[pl.ds(r * TOPK, 16)] = jnp.exp((kd0 - rmv) * TEMP) / dnv
        p_loc[pl.ds(r * TOPK + 16, 16)] = jnp.exp((kd1 - rmv) * TEMP) / dnv
        col = base + r
        gi_loc[pl.ds(r * TOPK, 16)] = id0 * F + col
        gi_loc[pl.ds(r * TOPK + 16, 16)] = id1 * F + col
        return 0

    lax.fori_loop(0, RPW, row_step, 0)

    handles = []
    for c in range(RPW * TOPK // 128):
        handles.append(pltpu.async_copy(
            wv_hbm.at[gi_loc.at[pl.ds(c * 128, 128)]],
            s_loc.at[pl.ds(c * 128, 128)], gsem))
    for h in handles:
        h.wait()

    pltpu.sync_copy(p_loc, p_hbm.at[pl.ds(base * TOPK, RPW * TOPK)])
    pltpu.sync_copy(s_loc, s_hbm.at[pl.ds(base * TOPK, RPW * TOPK)])


LQB = 256
LKB1 = 1024
LKB2 = 512


@jax.jit
def kernel(query, key_in, value, WQ, WK, WV):
    q2 = query[0]
    k2 = key_in[0]
    v2 = value[0]

    w_k = pl.pallas_call(
        _wk_body,
        grid=(LK // LKB1,),
        in_specs=[pl.BlockSpec((LKB1, HF), lambda i: (i, 0)),
                  pl.BlockSpec((QK, HF), lambda i: (0, 0))],
        out_specs=pl.BlockSpec((LKB1, QK), lambda i: (i, 0)),
        out_shape=jax.ShapeDtypeStruct((LK, QK), jnp.float32),
    )(k2, WK)

    scores, rmax, den = pl.pallas_call(
        _scores_body,
        grid=(LQ // LQB,),
        in_specs=[pl.BlockSpec((LQB, F), lambda i: (i, 0)),
                  pl.BlockSpec((QK, F), lambda i: (0, 0)),
                  pl.BlockSpec((LK, QK), lambda i: (0, 0))],
        out_specs=[pl.BlockSpec((LQB, LK), lambda i: (i, 0)),
                   pl.BlockSpec((LQB,), lambda i: (i,)),
                   pl.BlockSpec((LQB,), lambda i: (i,))],
        out_shape=[jax.ShapeDtypeStruct((LQ, LK), jnp.float32),
                   jax.ShapeDtypeStruct((LQ,), jnp.float32),
                   jax.ShapeDtypeStruct((LQ,), jnp.float32)],
    )(q2, WQ, w_k)

    w_v = pl.pallas_call(
        _wv_body,
        grid=(LK // LKB2,),
        in_specs=[pl.BlockSpec((LKB2, HF), lambda i: (i, 0)),
                  pl.BlockSpec((F, HF), lambda i: (0, 0))],
        out_specs=pl.BlockSpec((LKB2, F), lambda i: (i, 0)),
        out_shape=jax.ShapeDtypeStruct((LK, F), jnp.float32),
    )(v2, WV)

    sc_topk = functools.partial(
        pl.kernel,
        out_type=[jax.ShapeDtypeStruct((LQ * TOPK,), jnp.float32),
                  jax.ShapeDtypeStruct((LQ * TOPK,), jnp.float32)],
        mesh=plsc.VectorSubcoreMesh(core_axis_name="c", subcore_axis_name="s"),
        compiler_params=pltpu.CompilerParams(needs_layout_passes=False),
        scratch_types=[
            pltpu.VMEM((LK,), jnp.float32),
            pltpu.VMEM((NBINS,), jnp.int32),
            pltpu.VMEM((CAP + 16,), jnp.float32),
            pltpu.VMEM((CAP + 16,), jnp.int32),
            pltpu.VMEM((RPW,), jnp.float32),
            pltpu.VMEM((RPW,), jnp.float32),
            pltpu.VMEM((RPW * TOPK,), jnp.float32),
            pltpu.VMEM((RPW * TOPK,), jnp.float32),
            pltpu.VMEM((RPW * TOPK,), jnp.int32),
            pltpu.SemaphoreType.DMA,
        ],
    )(_sc_topk_body)

    p_flat, s_flat = sc_topk(scores.reshape(LQ * LK), rmax, den,
                             w_v.reshape(LK * F))
    P = p_flat.reshape(LQ, TOPK)
    S = s_flat.reshape(LQ, TOPK)

    out = pl.pallas_call(
        _final_body,
        grid=(LQ // LQB,),
        in_specs=[pl.BlockSpec((LQB, TOPK), lambda i: (i, 0)),
                  pl.BlockSpec((LQ, TOPK), lambda i: (0, 0)),
                  pl.BlockSpec((LQB,), lambda i: (i,)),
                  pl.BlockSpec((LQB, F), lambda i: (i, 0))],
        out_specs=pl.BlockSpec((LQB, F), lambda i: (i, 0)),
        out_shape=jax.ShapeDtypeStruct((LQ, F), jnp.float32),
    )(P, S, rmax, q2)

    return out[None]


# speedup vs baseline: 11.9062x; 2.8272x over previous
import functools

import jax
import jax.numpy as jnp
from jax import lax
from jax.experimental import pallas as pl
from jax.experimental.pallas import tpu as pltpu
from jax.experimental.pallas import tpu_sc as plsc

F = 2048
HF = 1024
QK = 64
LQ = 2048
LK = 8192
TOPK = 32
GATE = 0.1
TEMP = QK ** -0.5
EPS = 1e-12

NW = 32
RPW = LQ // NW
NBINS = 512
CAP = 1024


def _l2n(x):
    n = jnp.sqrt(jnp.sum(x * x, axis=1, keepdims=True))
    return x / jnp.maximum(n, EPS)


def _wk_body(kin_ref, wk_ref, o_ref):
    y = lax.dot_general(kin_ref[...], wk_ref[...], (((1,), (1,)), ((), ())),
                        preferred_element_type=jnp.float32)
    o_ref[...] = _l2n(y)


def _scores_body(q_ref, wq_ref, wk_ref, s_ref, m_ref, d_ref, t_ref):
    wq = lax.dot_general(q_ref[...], wq_ref[...], (((1,), (1,)), ((), ())),
                         preferred_element_type=jnp.float32)
    wq = _l2n(wq)
    s = lax.dot_general(wq, wk_ref[...], (((1,), (1,)), ((), ())),
                        preferred_element_type=jnp.float32)
    s_ref[...] = s
    m = jnp.max(s, axis=1)
    m_ref[...] = m
    d_ref[...] = jnp.sum(jnp.exp((s - m[:, None]) * TEMP), axis=1)
    mu = jnp.mean(s, axis=1)
    sg = jnp.sqrt(jnp.maximum(jnp.mean(s * s, axis=1) - mu * mu, 0.0))
    t_ref[...] = jnp.minimum(mu + 2.5 * sg, m)


def _wv_body(v_ref, wv_ref, o_ref):
    y = lax.dot_general(v_ref[...], wv_ref[...], (((1,), (1,)), ((), ())),
                        preferred_element_type=jnp.float32)
    o_ref[...] = _l2n(y)


def _final_body(p_ref, s_ref, m_ref, q_ref, o_ref):
    g = lax.dot_general(p_ref[...], s_ref[...], (((1,), (1,)), ((), ())),
                        preferred_element_type=jnp.float32)
    msk = (m_ref[...] > GATE).astype(jnp.float32)
    o_ref[...] = g * msk[:, None] + q_ref[...]


def _merge_keep_top(ak, ai, bk, bi):
    rbk = lax.rev(bk, (0,))
    rbi = lax.rev(bi, (0,))
    m = ak >= rbk
    hk = jnp.where(m, ak, rbk)
    hi = jnp.where(m, ai, rbi)
    return plsc.sort_key_val(hk, hi)


def _merge32(ak, ai, bk, bi):
    rbk = lax.rev(bk, (0,))
    rbi = lax.rev(bi, (0,))
    m = ak <= rbk
    lk_ = jnp.where(m, ak, rbk)
    li_ = jnp.where(m, ai, rbi)
    hk_ = jnp.where(m, rbk, ak)
    hi_ = jnp.where(m, rbi, ai)
    lk2, li2 = plsc.sort_key_val(lk_, li_)
    hk2, hi2 = plsc.sort_key_val(hk_, hi_)
    return lk2, li2, hk2, hi2


def _sc_topk_body(scores_hbm, rmax_hbm, den_hbm, thr_hbm, wv_hbm, p_hbm, s_hbm,
                  rowbuf, maxes, flag, hist, cval, cidx, rm_loc, dn_loc,
                  th_loc, p_loc, s_loc, gi_loc, rsem, gsem):
    wid = lax.axis_index("s") * 2 + lax.axis_index("c")
    base = wid * RPW
    pltpu.sync_copy(rmax_hbm.at[pl.ds(base, RPW)], rm_loc)
    pltpu.sync_copy(den_hbm.at[pl.ds(base, RPW)], dn_loc)
    pltpu.sync_copy(thr_hbm.at[pl.ds(base, RPW)], th_loc)
    iota16 = lax.iota(jnp.int32, 16)
    ones16 = jnp.ones((16,), jnp.int32)

    pltpu.async_copy(scores_hbm.at[pl.ds(base * LK, LK)],
                     rowbuf.at[pl.ds(0, LK)], rsem)

    def row_step(r, carry_unused):
        pbase = (r % 2) * LK
        buf = rowbuf.at[pl.ds(pbase, LK)]
        pltpu.make_async_copy(scores_hbm.at[pl.ds((base + r) * LK, LK)],
                              buf, rsem).wait()

        @pl.when(r + 1 < RPW)
        def _prefetch():
            pltpu.async_copy(scores_hbm.at[pl.ds((base + r + 1) * LK, LK)],
                             rowbuf.at[pl.ds(((r + 1) % 2) * LK, LK)], rsem)

        rsel = jnp.broadcast_to(r, (16,)).astype(jnp.int32)
        t0v = plsc.load_gather(th_loc, [rsel])

        def a_step(g, _):
            acc = jnp.zeros((16,), jnp.float32)
            for u in range(16):
                v = buf[pl.ds((g * 16 + u) * 16, 16)]
                acc = jnp.where(iota16 == u, jnp.max(v), acc)
            maxes[pl.ds(g * 16, 16)] = acc
            return 0

        lax.fori_loop(0, LK // 256, a_step, 0)

        def b_step(b, ptrf):
            mv = maxes[pl.ds(b * 16, 16)]
            m = mv >= t0v
            nf = jnp.max(plsc.all_reduce_population_count(m))
            plsc.store_compressed(flag.at[pl.ds(ptrf, 16)], b * 16 + iota16,
                                  mask=m)
            return ptrf + nf

        ptrf = lax.fori_loop(0, NBINS // 16, b_step, jnp.int32(0))

        def c_step(g, carry):
            ptr, tot = carry
            fv = flag[pl.ds(g * 16, 16)]
            for u in range(16):
                active = (g * 16 + u) < ptrf
                c = jnp.where(active, fv[u], 0)
                v = buf[pl.ds(c * 16, 16)]
                m = (v >= t0v) & active
                cnt = jnp.max(plsc.all_reduce_population_count(m))
                plsc.store_compressed(cval.at[pl.ds(ptr, 16)], v, mask=m)
                plsc.store_compressed(cidx.at[pl.ds(ptr, 16)],
                                      c * 16 + iota16, mask=m)
                ptr = jnp.minimum(ptr + cnt, CAP - 16)
                tot = tot + cnt
            return (ptr, tot)

        ptr, tot = lax.fori_loop(0, (ptrf + 15) // 16, c_step,
                                 (jnp.int32(0), jnp.int32(0)))

        def _fallback(_):
            for i in range(NBINS // 16):
                hist[pl.ds(i * 16, 16)] = jnp.zeros((16,), jnp.int32)

            def h_step(i, _):
                v = buf[pl.ds(i * 16, 16)]
                b = jnp.clip(((v + 1.0) * (NBINS / 2.0)).astype(jnp.int32),
                             0, NBINS - 1)
                plsc.addupdate_scatter(hist, [b], ones16)
                return 0

            lax.fori_loop(0, LK // 16, h_step, 0)

            def t_step(i, carry):
                cum, bfound, bbin = carry
                blk_id = (NBINS // 16 - 1) - i
                blk = hist[pl.ds(blk_id * 16, 16)]
                rcum = plsc.cumsum(lax.rev(blk, (0,))) + cum
                m = rcum >= TOPK
                any_m = jnp.max(m.astype(jnp.int32))
                ffs = jnp.max(plsc.all_reduce_ffs(m))
                cand_bin = blk_id * 16 + 15 - ffs
                bbin = jnp.where(bfound == 0,
                                 jnp.where(any_m == 1, cand_bin, bbin), bbin)
                bfound = jnp.maximum(bfound, any_m)
                return (cum + jnp.sum(blk), bfound, bbin)

            _, _, bbin = lax.fori_loop(
                0, NBINS // 16, t_step,
                (jnp.int32(0), jnp.int32(0), jnp.int32(0)))
            thresh = bbin.astype(jnp.float32) * (2.0 / NBINS) - 1.0

            def c2_step(i, p2):
                v = buf[pl.ds(i * 16, 16)]
                m = v >= thresh
                cnt = jnp.max(plsc.all_reduce_population_count(m))
                plsc.store_compressed(cval.at[pl.ds(p2, 16)], v, mask=m)
                plsc.store_compressed(cidx.at[pl.ds(p2, 16)], i * 16 + iota16,
                                      mask=m)
                return jnp.minimum(p2 + cnt, CAP - 16)

            return lax.fori_loop(0, LK // 16, c2_step, jnp.int32(0))

        ptr = lax.cond((tot < TOPK) | (tot > CAP - 16), _fallback,
                       lambda _: ptr, 0)
        cval[pl.ds(ptr, 16)] = jnp.full((16,), -2.0, jnp.float32)
        cidx[pl.ds(ptr, 16)] = jnp.zeros((16,), jnp.int32)

        def s_step(t, st):
            lok, loi, hik, hii = st
            ck = cval[pl.ds(t * 16, 16)]
            ci = cidx[pl.ds(t * 16, 16)]
            nk, ni = plsc.sort_key_val(ck, ci)
            h1k, h1i = _merge_keep_top(lok, loi, nk, ni)
            return _merge32(h1k, h1i, hik, hii)

        init = (jnp.full((16,), -3.0, jnp.float32), jnp.zeros((16,), jnp.int32),
                jnp.full((16,), -3.0, jnp.float32), jnp.zeros((16,), jnp.int32))
        lok, loi, hik, hii = lax.fori_loop(0, (ptr + 15) // 16, s_step, init)

        kd0 = lax.rev(hik, (0,))
        kd1 = lax.rev(lok, (0,))
        id0 = lax.rev(hii, (0,))
        id1 = lax.rev(loi, (0,))
        rmv = plsc.load_gather(rm_loc, [jnp.broadcast_to(r, (16,)).astype(jnp.int32)])
        dnv = plsc.load_gather(dn_loc, [jnp.broadcast_to(r, (16,)).astype(jnp.int32)])
        p_loc[pl.ds(r * TOPK, 16)] = jnp.exp((kd0 - rmv) * TEMP) / dnv
        p_loc[pl.ds(r * TOPK + 16, 16)] = jnp.exp((kd1 - rmv) * TEMP) / dnv
        col = base + r
        gi_loc[pl.ds(r * TOPK, 16)] = id0 * F + col
        gi_loc[pl.ds(r * TOPK + 16, 16)] = id1 * F + col
        return 0

    lax.fori_loop(0, RPW, row_step, 0)

    handles = []
    for c in range(RPW * TOPK // 128):
        handles.append(pltpu.async_copy(
            wv_hbm.at[gi_loc.at[pl.ds(c * 128, 128)]],
            s_loc.at[pl.ds(c * 128, 128)], gsem))
    for h in handles:
        h.wait()

    pltpu.sync_copy(p_loc, p_hbm.at[pl.ds(base * TOPK, RPW * TOPK)])
    pltpu.sync_copy(s_loc, s_hbm.at[pl.ds(base * TOPK, RPW * TOPK)])


LQB = 256
LKB1 = 1024
LKB2 = 512


@jax.jit
def kernel(query, key_in, value, WQ, WK, WV):
    q2 = query[0]
    k2 = key_in[0]
    v2 = value[0]

    w_k = pl.pallas_call(
        _wk_body,
        grid=(LK // LKB1,),
        in_specs=[pl.BlockSpec((LKB1, HF), lambda i: (i, 0)),
                  pl.BlockSpec((QK, HF), lambda i: (0, 0))],
        out_specs=pl.BlockSpec((LKB1, QK), lambda i: (i, 0)),
        out_shape=jax.ShapeDtypeStruct((LK, QK), jnp.float32),
    )(k2, WK)

    scores, rmax, den, thr = pl.pallas_call(
        _scores_body,
        grid=(LQ // LQB,),
        in_specs=[pl.BlockSpec((LQB, F), lambda i: (i, 0)),
                  pl.BlockSpec((QK, F), lambda i: (0, 0)),
                  pl.BlockSpec((LK, QK), lambda i: (0, 0))],
        out_specs=[pl.BlockSpec((LQB, LK), lambda i: (i, 0)),
                   pl.BlockSpec((LQB,), lambda i: (i,)),
                   pl.BlockSpec((LQB,), lambda i: (i,)),
                   pl.BlockSpec((LQB,), lambda i: (i,))],
        out_shape=[jax.ShapeDtypeStruct((LQ, LK), jnp.float32),
                   jax.ShapeDtypeStruct((LQ,), jnp.float32),
                   jax.ShapeDtypeStruct((LQ,), jnp.float32),
                   jax.ShapeDtypeStruct((LQ,), jnp.float32)],
    )(q2, WQ, w_k)

    w_v = pl.pallas_call(
        _wv_body,
        grid=(LK // LKB2,),
        in_specs=[pl.BlockSpec((LKB2, HF), lambda i: (i, 0)),
                  pl.BlockSpec((F, HF), lambda i: (0, 0))],
        out_specs=pl.BlockSpec((LKB2, F), lambda i: (i, 0)),
        out_shape=jax.ShapeDtypeStruct((LK, F), jnp.float32),
    )(v2, WV)

    sc_topk = functools.partial(
        pl.kernel,
        out_type=[jax.ShapeDtypeStruct((LQ * TOPK,), jnp.float32),
                  jax.ShapeDtypeStruct((LQ * TOPK,), jnp.float32)],
        mesh=plsc.VectorSubcoreMesh(core_axis_name="c", subcore_axis_name="s"),
        compiler_params=pltpu.CompilerParams(needs_layout_passes=False),
        scratch_types=[
            pltpu.VMEM((2 * LK,), jnp.float32),
            pltpu.VMEM((LK // 16,), jnp.float32),
            pltpu.VMEM((LK // 16 + 16,), jnp.int32),
            pltpu.VMEM((NBINS,), jnp.int32),
            pltpu.VMEM((CAP + 16,), jnp.float32),
            pltpu.VMEM((CAP + 16,), jnp.int32),
            pltpu.VMEM((RPW,), jnp.float32),
            pltpu.VMEM((RPW,), jnp.float32),
            pltpu.VMEM((RPW,), jnp.float32),
            pltpu.VMEM((RPW * TOPK,), jnp.float32),
            pltpu.VMEM((RPW * TOPK,), jnp.float32),
            pltpu.VMEM((RPW * TOPK,), jnp.int32),
            pltpu.SemaphoreType.DMA,
            pltpu.SemaphoreType.DMA,
        ],
    )(_sc_topk_body)

    p_flat, s_flat = sc_topk(scores.reshape(LQ * LK), rmax, den, thr,
                             w_v.reshape(LK * F))
    P = p_flat.reshape(LQ, TOPK)
    S = s_flat.reshape(LQ, TOPK)

    out = pl.pallas_call(
        _final_body,
        grid=(LQ // LQB,),
        in_specs=[pl.BlockSpec((LQB, TOPK), lambda i: (i, 0)),
                  pl.BlockSpec((LQ, TOPK), lambda i: (0, 0)),
                  pl.BlockSpec((LQB,), lambda i: (i,)),
                  pl.BlockSpec((LQB, F), lambda i: (i, 0))],
        out_specs=pl.BlockSpec((LQB, F), lambda i: (i, 0)),
        out_shape=jax.ShapeDtypeStruct((LQ, F), jnp.float32),
    )(P, S, rmax, q2)

    return out[None]


# speedup vs baseline: 13.9719x; 1.1735x over previous
import functools

import jax
import jax.numpy as jnp
from jax import lax
from jax.experimental import pallas as pl
from jax.experimental.pallas import tpu as pltpu
from jax.experimental.pallas import tpu_sc as plsc

F = 2048
HF = 1024
QK = 64
LQ = 2048
LK = 8192
TOPK = 32
GATE = 0.1
TEMP = QK ** -0.5
EPS = 1e-12

NW = 32
RPW = LQ // NW
NBINS = 512
CAP = 1024


def _l2n(x):
    n = jnp.sqrt(jnp.sum(x * x, axis=1, keepdims=True))
    return x / jnp.maximum(n, EPS)


def _wk_body(kin_ref, wk_ref, o_ref):
    y = lax.dot_general(kin_ref[...], wk_ref[...], (((1,), (1,)), ((), ())),
                        preferred_element_type=jnp.float32)
    o_ref[...] = _l2n(y)


def _scores_body(q_ref, wq_ref, wk_ref, s_ref, m_ref, d_ref, t_ref):
    wq = lax.dot_general(q_ref[...], wq_ref[...], (((1,), (1,)), ((), ())),
                         preferred_element_type=jnp.float32)
    wq = _l2n(wq)
    s = lax.dot_general(wq, wk_ref[...], (((1,), (1,)), ((), ())),
                        preferred_element_type=jnp.float32)
    s_ref[...] = s.reshape(LQB, LK // 128, 128)
    m = jnp.max(s, axis=1)
    m_ref[...] = m
    d_ref[...] = jnp.sum(jnp.exp((s - m[:, None]) * TEMP), axis=1)
    mu = jnp.mean(s, axis=1)
    sg = jnp.sqrt(jnp.maximum(jnp.mean(s * s, axis=1) - mu * mu, 0.0))
    t_ref[...] = jnp.minimum(mu + 2.5 * sg, m)


def _wv_body(v_ref, wv_ref, o_ref):
    y = lax.dot_general(v_ref[...], wv_ref[...], (((1,), (1,)), ((), ())),
                        preferred_element_type=jnp.float32)
    o_ref[...] = _l2n(y).reshape(LKB2, F // 128, 128)


def _final_body(p_ref, s_ref, m_ref, q_ref, o_ref):
    g = lax.dot_general(p_ref[...], s_ref[...], (((1,), (1,)), ((), ())),
                        preferred_element_type=jnp.float32)
    msk = (m_ref[...] > GATE).astype(jnp.float32)
    o_ref[...] = g * msk[:, None] + q_ref[...]


def _merge_keep_top(ak, ai, bk, bi):
    rbk = lax.rev(bk, (0,))
    rbi = lax.rev(bi, (0,))
    m = ak >= rbk
    hk = jnp.where(m, ak, rbk)
    hi = jnp.where(m, ai, rbi)
    return plsc.sort_key_val(hk, hi)


def _merge32(ak, ai, bk, bi):
    rbk = lax.rev(bk, (0,))
    rbi = lax.rev(bi, (0,))
    m = ak <= rbk
    lk_ = jnp.where(m, ak, rbk)
    li_ = jnp.where(m, ai, rbi)
    hk_ = jnp.where(m, rbk, ak)
    hi_ = jnp.where(m, rbi, ai)
    lk2, li2 = plsc.sort_key_val(lk_, li_)
    hk2, hi2 = plsc.sort_key_val(hk_, hi_)
    return lk2, li2, hk2, hi2


def _sc_topk_body(scores_hbm, rmax_hbm, den_hbm, thr_hbm, wv_hbm, p_hbm, s_hbm,
                  rowbuf, maxes, flag, hist, cval, cidx, rm_loc, dn_loc,
                  th_loc, p_loc, s_loc, gi_loc, rsem, gsem):
    wid = lax.axis_index("s") * 2 + lax.axis_index("c")
    base = wid * RPW
    pltpu.sync_copy(rmax_hbm.at[pl.ds(base, RPW)], rm_loc)
    pltpu.sync_copy(den_hbm.at[pl.ds(base, RPW)], dn_loc)
    pltpu.sync_copy(thr_hbm.at[pl.ds(base, RPW)], th_loc)
    iota16 = lax.iota(jnp.int32, 16)
    ones16 = jnp.ones((16,), jnp.int32)

    pltpu.async_copy(scores_hbm.at[pl.ds(base * LK, LK)],
                     rowbuf.at[pl.ds(0, LK)], rsem)

    def row_step(r, carry_unused):
        pbase = (r % 2) * LK
        buf = rowbuf.at[pl.ds(pbase, LK)]
        pltpu.make_async_copy(scores_hbm.at[pl.ds((base + r) * LK, LK)],
                              buf, rsem).wait()

        @pl.when(r + 1 < RPW)
        def _prefetch():
            pltpu.async_copy(scores_hbm.at[pl.ds((base + r + 1) * LK, LK)],
                             rowbuf.at[pl.ds(((r + 1) % 2) * LK, LK)], rsem)

        rsel = jnp.broadcast_to(r, (16,)).astype(jnp.int32)
        t0v = plsc.load_gather(th_loc, [rsel])

        def a_step(g, _):
            acc = jnp.zeros((16,), jnp.float32)
            for u in range(16):
                v = buf[pl.ds((g * 16 + u) * 16, 16)]
                acc = jnp.where(iota16 == u, jnp.max(v), acc)
            maxes[pl.ds(g * 16, 16)] = acc
            return 0

        lax.fori_loop(0, LK // 256, a_step, 0)

        def b_step(b, ptrf):
            mv = maxes[pl.ds(b * 16, 16)]
            m = mv >= t0v
            nf = jnp.max(plsc.all_reduce_population_count(m))
            plsc.store_compressed(flag.at[pl.ds(ptrf, 16)], b * 16 + iota16,
                                  mask=m)
            return ptrf + nf

        ptrf = lax.fori_loop(0, NBINS // 16, b_step, jnp.int32(0))

        def c_step(g, carry):
            ptr, tot = carry
            fv = flag[pl.ds(g * 16, 16)]
            for u in range(16):
                active = (g * 16 + u) < ptrf
                c = jnp.where(active, fv[u], 0)
                v = buf[pl.ds(c * 16, 16)]
                m = (v >= t0v) & active
                cnt = jnp.max(plsc.all_reduce_population_count(m))
                plsc.store_compressed(cval.at[pl.ds(ptr, 16)], v, mask=m)
                plsc.store_compressed(cidx.at[pl.ds(ptr, 16)],
                                      c * 16 + iota16, mask=m)
                ptr = jnp.minimum(ptr + cnt, CAP - 16)
                tot = tot + cnt
            return (ptr, tot)

        ptr, tot = lax.fori_loop(0, (ptrf + 15) // 16, c_step,
                                 (jnp.int32(0), jnp.int32(0)))

        def _fallback(_):
            for i in range(NBINS // 16):
                hist[pl.ds(i * 16, 16)] = jnp.zeros((16,), jnp.int32)

            def h_step(i, _):
                v = buf[pl.ds(i * 16, 16)]
                b = jnp.clip(((v + 1.0) * (NBINS / 2.0)).astype(jnp.int32),
                             0, NBINS - 1)
                plsc.addupdate_scatter(hist, [b], ones16)
                return 0

            lax.fori_loop(0, LK // 16, h_step, 0)

            def t_step(i, carry):
                cum, bfound, bbin = carry
                blk_id = (NBINS // 16 - 1) - i
                blk = hist[pl.ds(blk_id * 16, 16)]
                rcum = plsc.cumsum(lax.rev(blk, (0,))) + cum
                m = rcum >= TOPK
                any_m = jnp.max(m.astype(jnp.int32))
                ffs = jnp.max(plsc.all_reduce_ffs(m))
                cand_bin = blk_id * 16 + 15 - ffs
                bbin = jnp.where(bfound == 0,
                                 jnp.where(any_m == 1, cand_bin, bbin), bbin)
                bfound = jnp.maximum(bfound, any_m)
                return (cum + jnp.sum(blk), bfound, bbin)

            _, _, bbin = lax.fori_loop(
                0, NBINS // 16, t_step,
                (jnp.int32(0), jnp.int32(0), jnp.int32(0)))
            thresh = bbin.astype(jnp.float32) * (2.0 / NBINS) - 1.0

            def c2_step(i, p2):
                v = buf[pl.ds(i * 16, 16)]
                m = v >= thresh
                cnt = jnp.max(plsc.all_reduce_population_count(m))
                plsc.store_compressed(cval.at[pl.ds(p2, 16)], v, mask=m)
                plsc.store_compressed(cidx.at[pl.ds(p2, 16)], i * 16 + iota16,
                                      mask=m)
                return jnp.minimum(p2 + cnt, CAP - 16)

            return lax.fori_loop(0, LK // 16, c2_step, jnp.int32(0))

        ptr = lax.cond((tot < TOPK) | (tot > CAP - 16), _fallback,
                       lambda _: ptr, 0)
        cval[pl.ds(ptr, 16)] = jnp.full((16,), -2.0, jnp.float32)
        cidx[pl.ds(ptr, 16)] = jnp.zeros((16,), jnp.int32)

        def s_step(t, st):
            lok, loi, hik, hii = st
            ck = cval[pl.ds(t * 16, 16)]
            ci = cidx[pl.ds(t * 16, 16)]
            nk, ni = plsc.sort_key_val(ck, ci)
            h1k, h1i = _merge_keep_top(lok, loi, nk, ni)
            return _merge32(h1k, h1i, hik, hii)

        init = (jnp.full((16,), -3.0, jnp.float32), jnp.zeros((16,), jnp.int32),
                jnp.full((16,), -3.0, jnp.float32), jnp.zeros((16,), jnp.int32))
        lok, loi, hik, hii = lax.fori_loop(0, (ptr + 15) // 16, s_step, init)

        kd0 = lax.rev(hik, (0,))
        kd1 = lax.rev(lok, (0,))
        id0 = lax.rev(hii, (0,))
        id1 = lax.rev(loi, (0,))
        rmv = plsc.load_gather(rm_loc, [jnp.broadcast_to(r, (16,)).astype(jnp.int32)])
        dnv = plsc.load_gather(dn_loc, [jnp.broadcast_to(r, (16,)).astype(jnp.int32)])
        p_loc[pl.ds(r * TOPK, 16)] = jnp.exp((kd0 - rmv) * TEMP) / dnv
        p_loc[pl.ds(r * TOPK + 16, 16)] = jnp.exp((kd1 - rmv) * TEMP) / dnv
        col = base + r
        gi_loc[pl.ds(r * TOPK, 16)] = id0 * F + col
        gi_loc[pl.ds(r * TOPK + 16, 16)] = id1 * F + col
        return 0

    lax.fori_loop(0, RPW, row_step, 0)

    handles = []
    for c in range(RPW * TOPK // 128):
        handles.append(pltpu.async_copy(
            wv_hbm.at[gi_loc.at[pl.ds(c * 128, 128)]],
            s_loc.at[pl.ds(c * 128, 128)], gsem))
    for h in handles:
        h.wait()

    pltpu.sync_copy(p_loc, p_hbm.at[pl.ds(base * TOPK, RPW * TOPK)])
    pltpu.sync_copy(s_loc, s_hbm.at[pl.ds(base * TOPK, RPW * TOPK)])


LQB = 256
LKB1 = 1024
LKB2 = 512


@jax.jit
def kernel(query, key_in, value, WQ, WK, WV):
    q2 = query[0]
    k2 = key_in[0]
    v2 = value[0]

    w_k = pl.pallas_call(
        _wk_body,
        grid=(LK // LKB1,),
        in_specs=[pl.BlockSpec((LKB1, HF), lambda i: (i, 0)),
                  pl.BlockSpec((QK, HF), lambda i: (0, 0))],
        out_specs=pl.BlockSpec((LKB1, QK), lambda i: (i, 0)),
        out_shape=jax.ShapeDtypeStruct((LK, QK), jnp.float32),
    )(k2, WK)

    scores, rmax, den, thr = pl.pallas_call(
        _scores_body,
        grid=(LQ // LQB,),
        in_specs=[pl.BlockSpec((LQB, F), lambda i: (i, 0)),
                  pl.BlockSpec((QK, F), lambda i: (0, 0)),
                  pl.BlockSpec((LK, QK), lambda i: (0, 0))],
        out_specs=[pl.BlockSpec((LQB, LK // 128, 128), lambda i: (i, 0, 0)),
                   pl.BlockSpec((LQB,), lambda i: (i,)),
                   pl.BlockSpec((LQB,), lambda i: (i,)),
                   pl.BlockSpec((LQB,), lambda i: (i,))],
        out_shape=[jax.ShapeDtypeStruct((LQ, LK // 128, 128), jnp.float32),
                   jax.ShapeDtypeStruct((LQ,), jnp.float32),
                   jax.ShapeDtypeStruct((LQ,), jnp.float32),
                   jax.ShapeDtypeStruct((LQ,), jnp.float32)],
    )(q2, WQ, w_k)

    w_v = pl.pallas_call(
        _wv_body,
        grid=(LK // LKB2,),
        in_specs=[pl.BlockSpec((LKB2, HF), lambda i: (i, 0)),
                  pl.BlockSpec((F, HF), lambda i: (0, 0))],
        out_specs=pl.BlockSpec((LKB2, F // 128, 128), lambda i: (i, 0, 0)),
        out_shape=jax.ShapeDtypeStruct((LK, F // 128, 128), jnp.float32),
    )(v2, WV)

    sc_topk = functools.partial(
        pl.kernel,
        out_type=[jax.ShapeDtypeStruct((LQ * TOPK,), jnp.float32),
                  jax.ShapeDtypeStruct((LQ * TOPK,), jnp.float32)],
        mesh=plsc.VectorSubcoreMesh(core_axis_name="c", subcore_axis_name="s"),
        compiler_params=pltpu.CompilerParams(needs_layout_passes=False),
        scratch_types=[
            pltpu.VMEM((2 * LK,), jnp.float32),
            pltpu.VMEM((LK // 16,), jnp.float32),
            pltpu.VMEM((LK // 16 + 16,), jnp.int32),
            pltpu.VMEM((NBINS,), jnp.int32),
            pltpu.VMEM((CAP + 16,), jnp.float32),
            pltpu.VMEM((CAP + 16,), jnp.int32),
            pltpu.VMEM((RPW,), jnp.float32),
            pltpu.VMEM((RPW,), jnp.float32),
            pltpu.VMEM((RPW,), jnp.float32),
            pltpu.VMEM((RPW * TOPK,), jnp.float32),
            pltpu.VMEM((RPW * TOPK,), jnp.float32),
            pltpu.VMEM((RPW * TOPK,), jnp.int32),
            pltpu.SemaphoreType.DMA,
            pltpu.SemaphoreType.DMA,
        ],
    )(_sc_topk_body)

    p_flat, s_flat = sc_topk(scores.reshape(LQ * LK), rmax, den, thr,
                             w_v.reshape(LK * F))
    P = p_flat.reshape(LQ, TOPK)
    S = s_flat.reshape(LQ, TOPK)

    out = pl.pallas_call(
        _final_body,
        grid=(LQ // LQB,),
        in_specs=[pl.BlockSpec((LQB, TOPK), lambda i: (i, 0)),
                  pl.BlockSpec((LQ, TOPK), lambda i: (0, 0)),
                  pl.BlockSpec((LQB,), lambda i: (i,)),
                  pl.BlockSpec((LQB, F), lambda i: (i, 0))],
        out_specs=pl.BlockSpec((LQB, F), lambda i: (i, 0)),
        out_shape=jax.ShapeDtypeStruct((LQ, F), jnp.float32),
    )(P, S, rmax, q2)

    return out[None]


# speedup vs baseline: 14.5193x; 1.0392x over previous
import functools

import jax
import jax.numpy as jnp
from jax import lax
from jax.experimental import pallas as pl
from jax.experimental.pallas import tpu as pltpu
from jax.experimental.pallas import tpu_sc as plsc

F = 2048
HF = 1024
QK = 64
LQ = 2048
LK = 8192
TOPK = 32
GATE = 0.1
TEMP = QK ** -0.5
EPS = 1e-12

NW = 32
RPW = LQ // NW
NBINS = 512
CAP = 1024


def _l2n(x):
    n = jnp.sqrt(jnp.sum(x * x, axis=1, keepdims=True))
    return x / jnp.maximum(n, EPS)


def _wk_body(kin_ref, wk_ref, o_ref):
    y = lax.dot_general(kin_ref[...], wk_ref[...], (((1,), (1,)), ((), ())),
                        preferred_element_type=jnp.float32)
    o_ref[...] = _l2n(y)


def _scores_body(q_ref, wq_ref, wk_ref, s_ref, m_ref, d_ref, t_ref):
    wq = lax.dot_general(q_ref[...], wq_ref[...], (((1,), (1,)), ((), ())),
                         preferred_element_type=jnp.float32)
    wq = _l2n(wq)
    s = lax.dot_general(wq, wk_ref[...], (((1,), (1,)), ((), ())),
                        preferred_element_type=jnp.float32)
    s_ref[...] = s.reshape(LQB, LK // 128, 128)
    m = jnp.max(s, axis=1)
    m_ref[...] = m
    d_ref[...] = jnp.sum(jnp.exp((s - m[:, None]) * TEMP), axis=1)
    mu = jnp.mean(s, axis=1)
    sg = jnp.sqrt(jnp.maximum(jnp.mean(s * s, axis=1) - mu * mu, 0.0))
    t_ref[...] = jnp.minimum(mu + 2.5 * sg, m)


def _wv_body(v_ref, wv_ref, o_ref):
    y = lax.dot_general(v_ref[...], wv_ref[...], (((1,), (1,)), ((), ())),
                        preferred_element_type=jnp.float32)
    o_ref[...] = _l2n(y).reshape(LKB2, F // 128, 128)


def _final_body(p_ref, s_ref, m_ref, q_ref, o_ref):
    g = lax.dot_general(p_ref[...], s_ref[...], (((1,), (1,)), ((), ())),
                        preferred_element_type=jnp.float32)
    msk = (m_ref[...] > GATE).astype(jnp.float32)
    o_ref[...] = g * msk[:, None] + q_ref[...]


def _merge_keep_top(ak, ai, bk, bi):
    rbk = lax.rev(bk, (0,))
    rbi = lax.rev(bi, (0,))
    m = ak >= rbk
    hk = jnp.where(m, ak, rbk)
    hi = jnp.where(m, ai, rbi)
    return plsc.sort_key_val(hk, hi)


def _merge32(ak, ai, bk, bi):
    rbk = lax.rev(bk, (0,))
    rbi = lax.rev(bi, (0,))
    m = ak <= rbk
    lk_ = jnp.where(m, ak, rbk)
    li_ = jnp.where(m, ai, rbi)
    hk_ = jnp.where(m, rbk, ak)
    hi_ = jnp.where(m, rbi, ai)
    lk2, li2 = plsc.sort_key_val(lk_, li_)
    hk2, hi2 = plsc.sort_key_val(hk_, hi_)
    return lk2, li2, hk2, hi2


def _sc_topk_body(scores_hbm, rmax_hbm, den_hbm, thr_hbm, wv_hbm, p_hbm, s_hbm,
                  rowbuf, flag, hist, cval, cidx, rm_loc, dn_loc,
                  th_loc, p_loc, s_loc, gi_loc, rsem, gsem):
    wid = lax.axis_index("s") * 2 + lax.axis_index("c")
    base = wid * RPW
    pltpu.sync_copy(rmax_hbm.at[pl.ds(base, RPW)], rm_loc)
    pltpu.sync_copy(den_hbm.at[pl.ds(base, RPW)], dn_loc)
    pltpu.sync_copy(thr_hbm.at[pl.ds(base, RPW)], th_loc)
    iota16 = lax.iota(jnp.int32, 16)
    ones16 = jnp.ones((16,), jnp.int32)

    pltpu.async_copy(scores_hbm.at[pl.ds(base * LK, LK)],
                     rowbuf.at[pl.ds(0, LK)], rsem)

    def row_step(r, carry_unused):
        pbase = (r % 2) * LK
        buf = rowbuf.at[pl.ds(pbase, LK)]
        pltpu.make_async_copy(scores_hbm.at[pl.ds((base + r) * LK, LK)],
                              buf, rsem).wait()

        @pl.when(r + 1 < RPW)
        def _prefetch():
            pltpu.async_copy(scores_hbm.at[pl.ds((base + r + 1) * LK, LK)],
                             rowbuf.at[pl.ds(((r + 1) % 2) * LK, LK)], rsem)

        rsel = jnp.broadcast_to(r, (16,)).astype(jnp.int32)
        t0v = plsc.load_gather(th_loc, [rsel])

        def ab_step(g, ptrf):
            acc = jnp.zeros((16,), jnp.float32)
            for u in range(16):
                v = buf[pl.ds((g * 16 + u) * 16, 16)]
                acc = jnp.where(iota16 == u, jnp.max(v), acc)
            m = acc >= t0v
            nf = jnp.max(plsc.all_reduce_population_count(m))
            plsc.store_compressed(flag.at[pl.ds(ptrf, 16)], g * 16 + iota16,
                                  mask=m)
            return ptrf + nf

        ptrf = lax.fori_loop(0, LK // 256, ab_step, jnp.int32(0))

        def c_step(g, carry):
            ptr, tot = carry
            fv = flag[pl.ds(g * 16, 16)]
            for u in range(16):
                active = (g * 16 + u) < ptrf
                c = jnp.where(active, fv[u], 0)
                v = buf[pl.ds(c * 16, 16)]
                m = (v >= t0v) & active
                cnt = jnp.max(plsc.all_reduce_population_count(m))
                plsc.store_compressed(cval.at[pl.ds(ptr, 16)], v, mask=m)
                plsc.store_compressed(cidx.at[pl.ds(ptr, 16)],
                                      c * 16 + iota16, mask=m)
                ptr = jnp.minimum(ptr + cnt, CAP - 16)
                tot = tot + cnt
            return (ptr, tot)

        ptr, tot = lax.fori_loop(0, (ptrf + 15) // 16, c_step,
                                 (jnp.int32(0), jnp.int32(0)))

        def _fallback(_):
            for i in range(NBINS // 16):
                hist[pl.ds(i * 16, 16)] = jnp.zeros((16,), jnp.int32)

            def h_step(i, _):
                v = buf[pl.ds(i * 16, 16)]
                b = jnp.clip(((v + 1.0) * (NBINS / 2.0)).astype(jnp.int32),
                             0, NBINS - 1)
                plsc.addupdate_scatter(hist, [b], ones16)
                return 0

            lax.fori_loop(0, LK // 16, h_step, 0)

            def t_step(i, carry):
                cum, bfound, bbin = carry
                blk_id = (NBINS // 16 - 1) - i
                blk = hist[pl.ds(blk_id * 16, 16)]
                rcum = plsc.cumsum(lax.rev(blk, (0,))) + cum
                m = rcum >= TOPK
                any_m = jnp.max(m.astype(jnp.int32))
                ffs = jnp.max(plsc.all_reduce_ffs(m))
                cand_bin = blk_id * 16 + 15 - ffs
                bbin = jnp.where(bfound == 0,
                                 jnp.where(any_m == 1, cand_bin, bbin), bbin)
                bfound = jnp.maximum(bfound, any_m)
                return (cum + jnp.sum(blk), bfound, bbin)

            _, _, bbin = lax.fori_loop(
                0, NBINS // 16, t_step,
                (jnp.int32(0), jnp.int32(0), jnp.int32(0)))
            thresh = bbin.astype(jnp.float32) * (2.0 / NBINS) - 1.0

            def c2_step(i, p2):
                v = buf[pl.ds(i * 16, 16)]
                m = v >= thresh
                cnt = jnp.max(plsc.all_reduce_population_count(m))
                plsc.store_compressed(cval.at[pl.ds(p2, 16)], v, mask=m)
                plsc.store_compressed(cidx.at[pl.ds(p2, 16)], i * 16 + iota16,
                                      mask=m)
                return jnp.minimum(p2 + cnt, CAP - 16)

            return lax.fori_loop(0, LK // 16, c2_step, jnp.int32(0))

        ptr = lax.cond((tot < TOPK) | (tot > CAP - 16), _fallback,
                       lambda _: ptr, 0)
        cval[pl.ds(ptr, 16)] = jnp.full((16,), -2.0, jnp.float32)
        cidx[pl.ds(ptr, 16)] = jnp.zeros((16,), jnp.int32)

        def s_step(t, st):
            lok, loi, hik, hii = st
            ck = cval[pl.ds(t * 16, 16)]
            ci = cidx[pl.ds(t * 16, 16)]
            nk, ni = plsc.sort_key_val(ck, ci)
            h1k, h1i = _merge_keep_top(lok, loi, nk, ni)
            return _merge32(h1k, h1i, hik, hii)

        init = (jnp.full((16,), -3.0, jnp.float32), jnp.zeros((16,), jnp.int32),
                jnp.full((16,), -3.0, jnp.float32), jnp.zeros((16,), jnp.int32))
        lok, loi, hik, hii = lax.fori_loop(0, (ptr + 15) // 16, s_step, init)

        kd0 = lax.rev(hik, (0,))
        kd1 = lax.rev(lok, (0,))
        id0 = lax.rev(hii, (0,))
        id1 = lax.rev(loi, (0,))
        rmv = plsc.load_gather(rm_loc, [jnp.broadcast_to(r, (16,)).astype(jnp.int32)])
        dnv = plsc.load_gather(dn_loc, [jnp.broadcast_to(r, (16,)).astype(jnp.int32)])
        p_loc[pl.ds(r * TOPK, 16)] = jnp.exp((kd0 - rmv) * TEMP) / dnv
        p_loc[pl.ds(r * TOPK + 16, 16)] = jnp.exp((kd1 - rmv) * TEMP) / dnv
        col = base + r
        gi_loc[pl.ds(r * TOPK, 16)] = id0 * F + col
        gi_loc[pl.ds(r * TOPK + 16, 16)] = id1 * F + col
        return 0

    lax.fori_loop(0, RPW, row_step, 0)

    handles = []
    for c in range(RPW * TOPK // 128):
        handles.append(pltpu.async_copy(
            wv_hbm.at[gi_loc.at[pl.ds(c * 128, 128)]],
            s_loc.at[pl.ds(c * 128, 128)], gsem))
    for h in handles:
        h.wait()

    pltpu.sync_copy(p_loc, p_hbm.at[pl.ds(base * TOPK, RPW * TOPK)])
    pltpu.sync_copy(s_loc, s_hbm.at[pl.ds(base * TOPK, RPW * TOPK)])


LQB = 256
LKB1 = 1024
LKB2 = 512


@jax.jit
def kernel(query, key_in, value, WQ, WK, WV):
    q2 = query[0]
    k2 = key_in[0]
    v2 = value[0]

    w_k = pl.pallas_call(
        _wk_body,
        grid=(LK // LKB1,),
        in_specs=[pl.BlockSpec((LKB1, HF), lambda i: (i, 0)),
                  pl.BlockSpec((QK, HF), lambda i: (0, 0))],
        out_specs=pl.BlockSpec((LKB1, QK), lambda i: (i, 0)),
        out_shape=jax.ShapeDtypeStruct((LK, QK), jnp.float32),
    )(k2, WK)

    scores, rmax, den, thr = pl.pallas_call(
        _scores_body,
        grid=(LQ // LQB,),
        in_specs=[pl.BlockSpec((LQB, F), lambda i: (i, 0)),
                  pl.BlockSpec((QK, F), lambda i: (0, 0)),
                  pl.BlockSpec((LK, QK), lambda i: (0, 0))],
        out_specs=[pl.BlockSpec((LQB, LK // 128, 128), lambda i: (i, 0, 0)),
                   pl.BlockSpec((LQB,), lambda i: (i,)),
                   pl.BlockSpec((LQB,), lambda i: (i,)),
                   pl.BlockSpec((LQB,), lambda i: (i,))],
        out_shape=[jax.ShapeDtypeStruct((LQ, LK // 128, 128), jnp.float32),
                   jax.ShapeDtypeStruct((LQ,), jnp.float32),
                   jax.ShapeDtypeStruct((LQ,), jnp.float32),
                   jax.ShapeDtypeStruct((LQ,), jnp.float32)],
    )(q2, WQ, w_k)

    w_v = pl.pallas_call(
        _wv_body,
        grid=(LK // LKB2,),
        in_specs=[pl.BlockSpec((LKB2, HF), lambda i: (i, 0)),
                  pl.BlockSpec((F, HF), lambda i: (0, 0))],
        out_specs=pl.BlockSpec((LKB2, F // 128, 128), lambda i: (i, 0, 0)),
        out_shape=jax.ShapeDtypeStruct((LK, F // 128, 128), jnp.float32),
    )(v2, WV)

    sc_topk = functools.partial(
        pl.kernel,
        out_type=[jax.ShapeDtypeStruct((LQ * TOPK,), jnp.float32),
                  jax.ShapeDtypeStruct((LQ * TOPK,), jnp.float32)],
        mesh=plsc.VectorSubcoreMesh(core_axis_name="c", subcore_axis_name="s"),
        compiler_params=pltpu.CompilerParams(needs_layout_passes=False),
        scratch_types=[
            pltpu.VMEM((2 * LK,), jnp.float32),
            pltpu.VMEM((LK // 16 + 16,), jnp.int32),
            pltpu.VMEM((NBINS,), jnp.int32),
            pltpu.VMEM((CAP + 16,), jnp.float32),
            pltpu.VMEM((CAP + 16,), jnp.int32),
            pltpu.VMEM((RPW,), jnp.float32),
            pltpu.VMEM((RPW,), jnp.float32),
            pltpu.VMEM((RPW,), jnp.float32),
            pltpu.VMEM((RPW * TOPK,), jnp.float32),
            pltpu.VMEM((RPW * TOPK,), jnp.float32),
            pltpu.VMEM((RPW * TOPK,), jnp.int32),
            pltpu.SemaphoreType.DMA,
            pltpu.SemaphoreType.DMA,
        ],
    )(_sc_topk_body)

    p_flat, s_flat = sc_topk(scores.reshape(LQ * LK), rmax, den, thr,
                             w_v.reshape(LK * F))
    P = p_flat.reshape(LQ, TOPK)
    S = s_flat.reshape(LQ, TOPK)

    out = pl.pallas_call(
        _final_body,
        grid=(LQ // LQB,),
        in_specs=[pl.BlockSpec((LQB, TOPK), lambda i: (i, 0)),
                  pl.BlockSpec((LQ, TOPK), lambda i: (0, 0)),
                  pl.BlockSpec((LQB,), lambda i: (i,)),
                  pl.BlockSpec((LQB, F), lambda i: (i, 0))],
        out_specs=pl.BlockSpec((LQB, F), lambda i: (i, 0)),
        out_shape=jax.ShapeDtypeStruct((LQ, F), jnp.float32),
    )(P, S, rmax, q2)

    return out[None]


# speedup vs baseline: 14.8363x; 1.0218x over previous
import functools

import jax
import jax.numpy as jnp
from jax import lax
from jax.experimental import pallas as pl
from jax.experimental.pallas import tpu as pltpu
from jax.experimental.pallas import tpu_sc as plsc

F = 2048
HF = 1024
QK = 64
LQ = 2048
LK = 8192
TOPK = 32
GATE = 0.1
TEMP = QK ** -0.5
EPS = 1e-12

NW = 32
RPW = LQ // NW
NBINS = 512
CAP = 1024


def _l2n(x):
    n = jnp.sqrt(jnp.sum(x * x, axis=1, keepdims=True))
    return x / jnp.maximum(n, EPS)


def _wk_body(kin_ref, wk_ref, o_ref):
    y = lax.dot_general(kin_ref[...], wk_ref[...], (((1,), (1,)), ((), ())),
                        preferred_element_type=jnp.float32)
    o_ref[...] = _l2n(y)


def _scores_body(q_ref, wq_ref, wk_ref, s_ref, m_ref, d_ref, t_ref):
    wq = lax.dot_general(q_ref[...], wq_ref[...], (((1,), (1,)), ((), ())),
                         preferred_element_type=jnp.float32)
    wq = _l2n(wq)
    s = lax.dot_general(wq, wk_ref[...], (((1,), (1,)), ((), ())),
                        preferred_element_type=jnp.float32)
    s_ref[...] = s.reshape(LQB, LK // 128, 128)
    m = jnp.max(s, axis=1)
    m_ref[...] = m
    d_ref[...] = jnp.sum(jnp.exp((s - m[:, None]) * TEMP), axis=1)
    mu = jnp.mean(s, axis=1)
    sg = jnp.sqrt(jnp.maximum(jnp.mean(s * s, axis=1) - mu * mu, 0.0))
    t_ref[...] = jnp.minimum(mu + 2.5 * sg, m)


def _wv_body(v_ref, wv_ref, o_ref):
    y = lax.dot_general(v_ref[...], wv_ref[...], (((1,), (1,)), ((), ())),
                        preferred_element_type=jnp.float32)
    o_ref[...] = _l2n(y).reshape(LKB2, F // 128, 128)


def _final_body(p_ref, s_ref, m_ref, q_ref, o_ref):
    g = lax.dot_general(p_ref[...], s_ref[...], (((1,), (1,)), ((), ())),
                        preferred_element_type=jnp.float32)
    msk = (m_ref[...] > GATE).astype(jnp.float32)
    o_ref[...] = g * msk[:, None] + q_ref[...]


def _merge_keep_top(ak, ai, bk, bi):
    rbk = lax.rev(bk, (0,))
    rbi = lax.rev(bi, (0,))
    m = ak >= rbk
    hk = jnp.where(m, ak, rbk)
    hi = jnp.where(m, ai, rbi)
    return plsc.sort_key_val(hk, hi)


def _merge32(ak, ai, bk, bi):
    rbk = lax.rev(bk, (0,))
    rbi = lax.rev(bi, (0,))
    m = ak <= rbk
    lk_ = jnp.where(m, ak, rbk)
    li_ = jnp.where(m, ai, rbi)
    hk_ = jnp.where(m, rbk, ak)
    hi_ = jnp.where(m, rbi, ai)
    lk2, li2 = plsc.sort_key_val(lk_, li_)
    hk2, hi2 = plsc.sort_key_val(hk_, hi_)
    return lk2, li2, hk2, hi2


def _sc_topk_body(scores_hbm, rmax_hbm, den_hbm, thr_hbm, wv_hbm, p_hbm, s_hbm,
                  rowbuf, flag, hist, cval, cidx, rm_loc, dn_loc,
                  th_loc, p_loc, s_loc, gi_loc, rsem, gsem):
    wid = lax.axis_index("s") * 2 + lax.axis_index("c")
    base = wid * RPW
    pltpu.sync_copy(rmax_hbm.at[pl.ds(base, RPW)], rm_loc)
    pltpu.sync_copy(den_hbm.at[pl.ds(base, RPW)], dn_loc)
    pltpu.sync_copy(thr_hbm.at[pl.ds(base, RPW)], th_loc)
    iota16 = lax.iota(jnp.int32, 16)
    ones16 = jnp.ones((16,), jnp.int32)

    pltpu.async_copy(scores_hbm.at[pl.ds(base * LK, LK)],
                     rowbuf.at[pl.ds(0, LK)], rsem)

    def row_step(r, carry_unused):
        pbase = (r % 2) * LK
        buf = rowbuf.at[pl.ds(pbase, LK)]
        pltpu.make_async_copy(scores_hbm.at[pl.ds((base + r) * LK, LK)],
                              buf, rsem).wait()

        @pl.when(r + 1 < RPW)
        def _prefetch():
            pltpu.async_copy(scores_hbm.at[pl.ds((base + r + 1) * LK, LK)],
                             rowbuf.at[pl.ds(((r + 1) % 2) * LK, LK)], rsem)

        rsel = jnp.broadcast_to(r, (16,)).astype(jnp.int32)
        t0v = plsc.load_gather(th_loc, [rsel])

        def ab_step(g, ptrf):
            acc = jnp.zeros((16,), jnp.float32)
            for u in range(16):
                cb = (g * 16 + u) * 32
                v = jnp.maximum(buf[pl.ds(cb, 16)], buf[pl.ds(cb + 16, 16)])
                acc = jnp.where(iota16 == u, jnp.max(v), acc)
            m = acc >= t0v
            nf = plsc.all_reduce_population_count(m)[0]
            plsc.store_compressed(flag.at[pl.ds(ptrf, 16)], g * 16 + iota16,
                                  mask=m)
            return ptrf + nf

        ptrf = lax.fori_loop(0, LK // 512, ab_step, jnp.int32(0))

        def c_step(g, carry):
            ptr, tot = carry
            fv = flag[pl.ds(g * 16, 16)]
            for u in range(16):
                active = (g * 16 + u) < ptrf
                c = jnp.where(active, fv[u], 0)
                for h in range(2):
                    v = buf[pl.ds(c * 32 + h * 16, 16)]
                    m = (v >= t0v) & active
                    cnt = plsc.all_reduce_population_count(m)[0]
                    plsc.store_compressed(cval.at[pl.ds(ptr, 16)], v, mask=m)
                    plsc.store_compressed(cidx.at[pl.ds(ptr, 16)],
                                          c * 32 + h * 16 + iota16, mask=m)
                    ptr = jnp.minimum(ptr + cnt, CAP - 16)
                    tot = tot + cnt
            return (ptr, tot)

        ptr, tot = lax.fori_loop(0, (ptrf + 15) // 16, c_step,
                                 (jnp.int32(0), jnp.int32(0)))

        def _fallback(_):
            for i in range(NBINS // 16):
                hist[pl.ds(i * 16, 16)] = jnp.zeros((16,), jnp.int32)

            def h_step(i, _):
                v = buf[pl.ds(i * 16, 16)]
                b = jnp.clip(((v + 1.0) * (NBINS / 2.0)).astype(jnp.int32),
                             0, NBINS - 1)
                plsc.addupdate_scatter(hist, [b], ones16)
                return 0

            lax.fori_loop(0, LK // 16, h_step, 0)

            def t_step(i, carry):
                cum, bfound, bbin = carry
                blk_id = (NBINS // 16 - 1) - i
                blk = hist[pl.ds(blk_id * 16, 16)]
                rcum = plsc.cumsum(lax.rev(blk, (0,))) + cum
                m = rcum >= TOPK
                any_m = jnp.max(m.astype(jnp.int32))
                ffs = jnp.max(plsc.all_reduce_ffs(m))
                cand_bin = blk_id * 16 + 15 - ffs
                bbin = jnp.where(bfound == 0,
                                 jnp.where(any_m == 1, cand_bin, bbin), bbin)
                bfound = jnp.maximum(bfound, any_m)
                return (cum + jnp.sum(blk), bfound, bbin)

            _, _, bbin = lax.fori_loop(
                0, NBINS // 16, t_step,
                (jnp.int32(0), jnp.int32(0), jnp.int32(0)))
            thresh = bbin.astype(jnp.float32) * (2.0 / NBINS) - 1.0

            def c2_step(i, p2):
                v = buf[pl.ds(i * 16, 16)]
                m = v >= thresh
                cnt = plsc.all_reduce_population_count(m)[0]
                plsc.store_compressed(cval.at[pl.ds(p2, 16)], v, mask=m)
                plsc.store_compressed(cidx.at[pl.ds(p2, 16)], i * 16 + iota16,
                                      mask=m)
                return jnp.minimum(p2 + cnt, CAP - 16)

            return lax.fori_loop(0, LK // 16, c2_step, jnp.int32(0))

        ptr = lax.cond((tot < TOPK) | (tot > CAP - 16), _fallback,
                       lambda _: ptr, 0)
        cval[pl.ds(ptr, 16)] = jnp.full((16,), -2.0, jnp.float32)
        cidx[pl.ds(ptr, 16)] = jnp.zeros((16,), jnp.int32)

        def s_step(t, st):
            lok, loi, hik, hii = st
            ck = cval[pl.ds(t * 16, 16)]
            ci = cidx[pl.ds(t * 16, 16)]
            nk, ni = plsc.sort_key_val(ck, ci)
            h1k, h1i = _merge_keep_top(lok, loi, nk, ni)
            return _merge32(h1k, h1i, hik, hii)

        init = (jnp.full((16,), -3.0, jnp.float32), jnp.zeros((16,), jnp.int32),
                jnp.full((16,), -3.0, jnp.float32), jnp.zeros((16,), jnp.int32))
        lok, loi, hik, hii = lax.fori_loop(0, (ptr + 15) // 16, s_step, init)

        kd0 = lax.rev(hik, (0,))
        kd1 = lax.rev(lok, (0,))
        id0 = lax.rev(hii, (0,))
        id1 = lax.rev(loi, (0,))
        rmv = plsc.load_gather(rm_loc, [jnp.broadcast_to(r, (16,)).astype(jnp.int32)])
        dnv = plsc.load_gather(dn_loc, [jnp.broadcast_to(r, (16,)).astype(jnp.int32)])
        p_loc[pl.ds(r * TOPK, 16)] = jnp.exp((kd0 - rmv) * TEMP) / dnv
        p_loc[pl.ds(r * TOPK + 16, 16)] = jnp.exp((kd1 - rmv) * TEMP) / dnv
        col = base + r
        gi_loc[pl.ds(r * TOPK, 16)] = id0 * F + col
        gi_loc[pl.ds(r * TOPK + 16, 16)] = id1 * F + col
        return 0

    lax.fori_loop(0, RPW, row_step, 0)

    handles = []
    for c in range(RPW * TOPK // 128):
        handles.append(pltpu.async_copy(
            wv_hbm.at[gi_loc.at[pl.ds(c * 128, 128)]],
            s_loc.at[pl.ds(c * 128, 128)], gsem))
    for h in handles:
        h.wait()

    pltpu.sync_copy(p_loc, p_hbm.at[pl.ds(base * TOPK, RPW * TOPK)])
    pltpu.sync_copy(s_loc, s_hbm.at[pl.ds(base * TOPK, RPW * TOPK)])


LQB = 256
LKB1 = 1024
LKB2 = 512


@jax.jit
def kernel(query, key_in, value, WQ, WK, WV):
    q2 = query[0]
    k2 = key_in[0]
    v2 = value[0]

    w_k = pl.pallas_call(
        _wk_body,
        grid=(LK // LKB1,),
        in_specs=[pl.BlockSpec((LKB1, HF), lambda i: (i, 0)),
                  pl.BlockSpec((QK, HF), lambda i: (0, 0))],
        out_specs=pl.BlockSpec((LKB1, QK), lambda i: (i, 0)),
        out_shape=jax.ShapeDtypeStruct((LK, QK), jnp.float32),
    )(k2, WK)

    scores, rmax, den, thr = pl.pallas_call(
        _scores_body,
        grid=(LQ // LQB,),
        in_specs=[pl.BlockSpec((LQB, F), lambda i: (i, 0)),
                  pl.BlockSpec((QK, F), lambda i: (0, 0)),
                  pl.BlockSpec((LK, QK), lambda i: (0, 0))],
        out_specs=[pl.BlockSpec((LQB, LK // 128, 128), lambda i: (i, 0, 0)),
                   pl.BlockSpec((LQB,), lambda i: (i,)),
                   pl.BlockSpec((LQB,), lambda i: (i,)),
                   pl.BlockSpec((LQB,), lambda i: (i,))],
        out_shape=[jax.ShapeDtypeStruct((LQ, LK // 128, 128), jnp.float32),
                   jax.ShapeDtypeStruct((LQ,), jnp.float32),
                   jax.ShapeDtypeStruct((LQ,), jnp.float32),
                   jax.ShapeDtypeStruct((LQ,), jnp.float32)],
    )(q2, WQ, w_k)

    w_v = pl.pallas_call(
        _wv_body,
        grid=(LK // LKB2,),
        in_specs=[pl.BlockSpec((LKB2, HF), lambda i: (i, 0)),
                  pl.BlockSpec((F, HF), lambda i: (0, 0))],
        out_specs=pl.BlockSpec((LKB2, F // 128, 128), lambda i: (i, 0, 0)),
        out_shape=jax.ShapeDtypeStruct((LK, F // 128, 128), jnp.float32),
    )(v2, WV)

    sc_topk = functools.partial(
        pl.kernel,
        out_type=[jax.ShapeDtypeStruct((LQ * TOPK,), jnp.float32),
                  jax.ShapeDtypeStruct((LQ * TOPK,), jnp.float32)],
        mesh=plsc.VectorSubcoreMesh(core_axis_name="c", subcore_axis_name="s"),
        compiler_params=pltpu.CompilerParams(needs_layout_passes=False),
        scratch_types=[
            pltpu.VMEM((2 * LK,), jnp.float32),
            pltpu.VMEM((LK // 16 + 16,), jnp.int32),
            pltpu.VMEM((NBINS,), jnp.int32),
            pltpu.VMEM((CAP + 16,), jnp.float32),
            pltpu.VMEM((CAP + 16,), jnp.int32),
            pltpu.VMEM((RPW,), jnp.float32),
            pltpu.VMEM((RPW,), jnp.float32),
            pltpu.VMEM((RPW,), jnp.float32),
            pltpu.VMEM((RPW * TOPK,), jnp.float32),
            pltpu.VMEM((RPW * TOPK,), jnp.float32),
            pltpu.VMEM((RPW * TOPK,), jnp.int32),
            pltpu.SemaphoreType.DMA,
            pltpu.SemaphoreType.DMA,
        ],
    )(_sc_topk_body)

    p_flat, s_flat = sc_topk(scores.reshape(LQ * LK), rmax, den, thr,
                             w_v.reshape(LK * F))
    P = p_flat.reshape(LQ, TOPK)
    S = s_flat.reshape(LQ, TOPK)

    out = pl.pallas_call(
        _final_body,
        grid=(LQ // LQB,),
        in_specs=[pl.BlockSpec((LQB, TOPK), lambda i: (i, 0)),
                  pl.BlockSpec((LQ, TOPK), lambda i: (0, 0)),
                  pl.BlockSpec((LQB,), lambda i: (i,)),
                  pl.BlockSpec((LQB, F), lambda i: (i, 0))],
        out_specs=pl.BlockSpec((LQB, F), lambda i: (i, 0)),
        out_shape=jax.ShapeDtypeStruct((LQ, F), jnp.float32),
    )(P, S, rmax, q2)

    return out[None]


# speedup vs baseline: 17.9676x; 1.2111x over previous
import functools

import jax
import jax.numpy as jnp
from jax import lax
from jax.experimental import pallas as pl
from jax.experimental.pallas import tpu as pltpu
from jax.experimental.pallas import tpu_sc as plsc

F = 2048
HF = 1024
QK = 64
LQ = 2048
LK = 8192
TOPK = 32
GATE = 0.1
TEMP = QK ** -0.5
EPS = 1e-12

NW = 32
RPW = LQ // NW
NBINS = 512
CAP = 1024


def _l2n(x):
    n = jnp.sqrt(jnp.sum(x * x, axis=1, keepdims=True))
    return x / jnp.maximum(n, EPS)


def _wk_body(kin_ref, wk_ref, o_ref):
    y = lax.dot_general(kin_ref[...], wk_ref[...], (((1,), (1,)), ((), ())),
                        preferred_element_type=jnp.float32)
    o_ref[...] = _l2n(y)


def _scores_body(q_ref, wq_ref, wk_ref, s_ref, m_ref, d_ref, t_ref):
    wq = lax.dot_general(q_ref[...], wq_ref[...], (((1,), (1,)), ((), ())),
                         preferred_element_type=jnp.float32)
    wq = _l2n(wq)
    s = lax.dot_general(wq, wk_ref[...], (((1,), (1,)), ((), ())),
                        preferred_element_type=jnp.float32)
    s_ref[...] = s.reshape(LQB, LK // 128, 128)
    m = jnp.max(s, axis=1)
    m_ref[...] = m
    d_ref[...] = jnp.sum(jnp.exp((s - m[:, None]) * TEMP), axis=1)
    mu = jnp.mean(s, axis=1)
    sg = jnp.sqrt(jnp.maximum(jnp.mean(s * s, axis=1) - mu * mu, 0.0))
    t_ref[...] = jnp.minimum(mu + 2.5 * sg, m)


def _wv_body(v_ref, wv_ref, o_ref):
    y = lax.dot_general(v_ref[...], wv_ref[...], (((1,), (1,)), ((), ())),
                        preferred_element_type=jnp.float32)
    o_ref[...] = _l2n(y).reshape(LKB2, F // 128, 128)


def _final_body(p_ref, s_ref, m_ref, q_ref, o_ref):
    g = lax.dot_general(p_ref[...], s_ref[...], (((1,), (1,)), ((), ())),
                        preferred_element_type=jnp.float32)
    msk = (m_ref[...] > GATE).astype(jnp.float32)
    o_ref[...] = g * msk[:, None] + q_ref[...]


def _merge_keep_top(ak, ai, bk, bi):
    rbk = lax.rev(bk, (0,))
    rbi = lax.rev(bi, (0,))
    m = ak >= rbk
    hk = jnp.where(m, ak, rbk)
    hi = jnp.where(m, ai, rbi)
    return plsc.sort_key_val(hk, hi)


def _merge32(ak, ai, bk, bi):
    rbk = lax.rev(bk, (0,))
    rbi = lax.rev(bi, (0,))
    m = ak <= rbk
    lk_ = jnp.where(m, ak, rbk)
    li_ = jnp.where(m, ai, rbi)
    hk_ = jnp.where(m, rbk, ak)
    hi_ = jnp.where(m, rbi, ai)
    lk2, li2 = plsc.sort_key_val(lk_, li_)
    hk2, hi2 = plsc.sort_key_val(hk_, hi_)
    return lk2, li2, hk2, hi2


def _sc_topk_body(scores_hbm, rmax_hbm, den_hbm, thr_hbm, p_hbm, gi_hbm,
                  rowbuf, flag, hist, cval, cidx, rm_loc, dn_loc,
                  th_loc, p_loc, gi_loc, rsem):
    wid = lax.axis_index("s") * 2 + lax.axis_index("c")
    base = wid * RPW
    pltpu.sync_copy(rmax_hbm.at[pl.ds(base, RPW)], rm_loc)
    pltpu.sync_copy(den_hbm.at[pl.ds(base, RPW)], dn_loc)
    pltpu.sync_copy(thr_hbm.at[pl.ds(base, RPW)], th_loc)
    iota16 = lax.iota(jnp.int32, 16)
    ones16 = jnp.ones((16,), jnp.int32)

    pltpu.async_copy(scores_hbm.at[pl.ds(base * LK, LK)],
                     rowbuf.at[pl.ds(0, LK)], rsem)

    def row_step(r, carry_unused):
        pbase = (r % 2) * LK
        buf = rowbuf.at[pl.ds(pbase, LK)]
        pltpu.make_async_copy(scores_hbm.at[pl.ds((base + r) * LK, LK)],
                              buf, rsem).wait()

        @pl.when(r + 1 < RPW)
        def _prefetch():
            pltpu.async_copy(scores_hbm.at[pl.ds((base + r + 1) * LK, LK)],
                             rowbuf.at[pl.ds(((r + 1) % 2) * LK, LK)], rsem)

        rsel = jnp.broadcast_to(r, (16,)).astype(jnp.int32)
        t0v = plsc.load_gather(th_loc, [rsel])

        def ab_step(g, ptrf):
            acc = jnp.zeros((16,), jnp.float32)
            for u in range(16):
                cb = (g * 16 + u) * 32
                v = jnp.maximum(buf[pl.ds(cb, 16)], buf[pl.ds(cb + 16, 16)])
                acc = jnp.where(iota16 == u, jnp.max(v), acc)
            m = acc >= t0v
            nf = plsc.all_reduce_population_count(m)[0]
            plsc.store_compressed(flag.at[pl.ds(ptrf, 16)], g * 16 + iota16,
                                  mask=m)
            return ptrf + nf

        ptrf = lax.fori_loop(0, LK // 512, ab_step, jnp.int32(0))

        def c_step(g, carry):
            ptr, tot = carry
            fv = flag[pl.ds(g * 16, 16)]
            for u in range(16):
                active = (g * 16 + u) < ptrf
                c = jnp.where(active, fv[u], 0)
                for h in range(2):
                    v = buf[pl.ds(c * 32 + h * 16, 16)]
                    m = (v >= t0v) & active
                    cnt = plsc.all_reduce_population_count(m)[0]
                    plsc.store_compressed(cval.at[pl.ds(ptr, 16)], v, mask=m)
                    plsc.store_compressed(cidx.at[pl.ds(ptr, 16)],
                                          c * 32 + h * 16 + iota16, mask=m)
                    ptr = jnp.minimum(ptr + cnt, CAP - 16)
                    tot = tot + cnt
            return (ptr, tot)

        ptr, tot = lax.fori_loop(0, (ptrf + 15) // 16, c_step,
                                 (jnp.int32(0), jnp.int32(0)))

        def _fallback(_):
            for i in range(NBINS // 16):
                hist[pl.ds(i * 16, 16)] = jnp.zeros((16,), jnp.int32)

            def h_step(i, _):
                v = buf[pl.ds(i * 16, 16)]
                b = jnp.clip(((v + 1.0) * (NBINS / 2.0)).astype(jnp.int32),
                             0, NBINS - 1)
                plsc.addupdate_scatter(hist, [b], ones16)
                return 0

            lax.fori_loop(0, LK // 16, h_step, 0)

            def t_step(i, carry):
                cum, bfound, bbin = carry
                blk_id = (NBINS // 16 - 1) - i
                blk = hist[pl.ds(blk_id * 16, 16)]
                rcum = plsc.cumsum(lax.rev(blk, (0,))) + cum
                m = rcum >= TOPK
                any_m = jnp.max(m.astype(jnp.int32))
                ffs = jnp.max(plsc.all_reduce_ffs(m))
                cand_bin = blk_id * 16 + 15 - ffs
                bbin = jnp.where(bfound == 0,
                                 jnp.where(any_m == 1, cand_bin, bbin), bbin)
                bfound = jnp.maximum(bfound, any_m)
                return (cum + jnp.sum(blk), bfound, bbin)

            _, _, bbin = lax.fori_loop(
                0, NBINS // 16, t_step,
                (jnp.int32(0), jnp.int32(0), jnp.int32(0)))
            thresh = bbin.astype(jnp.float32) * (2.0 / NBINS) - 1.0

            def c2_step(i, p2):
                v = buf[pl.ds(i * 16, 16)]
                m = v >= thresh
                cnt = plsc.all_reduce_population_count(m)[0]
                plsc.store_compressed(cval.at[pl.ds(p2, 16)], v, mask=m)
                plsc.store_compressed(cidx.at[pl.ds(p2, 16)], i * 16 + iota16,
                                      mask=m)
                return jnp.minimum(p2 + cnt, CAP - 16)

            return lax.fori_loop(0, LK // 16, c2_step, jnp.int32(0))

        ptr = lax.cond((tot < TOPK) | (tot > CAP - 16), _fallback,
                       lambda _: ptr, 0)
        cval[pl.ds(ptr, 16)] = jnp.full((16,), -2.0, jnp.float32)
        cidx[pl.ds(ptr, 16)] = jnp.zeros((16,), jnp.int32)

        def s_step(t, st):
            lok, loi, hik, hii = st
            ck = cval[pl.ds(t * 16, 16)]
            ci = cidx[pl.ds(t * 16, 16)]
            nk, ni = plsc.sort_key_val(ck, ci)
            h1k, h1i = _merge_keep_top(lok, loi, nk, ni)
            return _merge32(h1k, h1i, hik, hii)

        init = (jnp.full((16,), -3.0, jnp.float32), jnp.zeros((16,), jnp.int32),
                jnp.full((16,), -3.0, jnp.float32), jnp.zeros((16,), jnp.int32))
        lok, loi, hik, hii = lax.fori_loop(0, (ptr + 15) // 16, s_step, init)

        kd0 = lax.rev(hik, (0,))
        kd1 = lax.rev(lok, (0,))
        id0 = lax.rev(hii, (0,))
        id1 = lax.rev(loi, (0,))
        rmv = plsc.load_gather(rm_loc, [jnp.broadcast_to(r, (16,)).astype(jnp.int32)])
        dnv = plsc.load_gather(dn_loc, [jnp.broadcast_to(r, (16,)).astype(jnp.int32)])
        p_loc[pl.ds(r * TOPK, 16)] = jnp.exp((kd0 - rmv) * TEMP) / dnv
        p_loc[pl.ds(r * TOPK + 16, 16)] = jnp.exp((kd1 - rmv) * TEMP) / dnv
        col = base + r
        gi_loc[pl.ds(r * TOPK, 16)] = id0 * F + col
        gi_loc[pl.ds(r * TOPK + 16, 16)] = id1 * F + col
        return 0

    lax.fori_loop(0, RPW, row_step, 0)

    pltpu.sync_copy(p_loc, p_hbm.at[pl.ds(base * TOPK, RPW * TOPK)])
    pltpu.sync_copy(gi_loc, gi_hbm.at[pl.ds(base * TOPK, RPW * TOPK)])


def _sc_gather_body(gi_hbm, wv_hbm, s_hbm, gi_loc, s_loc, gsem):
    wid = lax.axis_index("s") * 2 + lax.axis_index("c")
    base = wid * RPW
    pltpu.sync_copy(gi_hbm.at[pl.ds(base * TOPK, RPW * TOPK)], gi_loc)
    handles = []
    for c in range(RPW * TOPK // 128):
        handles.append(pltpu.async_copy(
            wv_hbm.at[gi_loc.at[pl.ds(c * 128, 128)]],
            s_loc.at[pl.ds(c * 128, 128)], gsem))
    for h in handles:
        h.wait()
    pltpu.sync_copy(s_loc, s_hbm.at[pl.ds(base * TOPK, RPW * TOPK)])


LQB = 256
LKB1 = 1024
LKB2 = 512


@jax.jit
def kernel(query, key_in, value, WQ, WK, WV):
    q2 = query[0]
    k2 = key_in[0]
    v2 = value[0]

    w_k = pl.pallas_call(
        _wk_body,
        grid=(LK // LKB1,),
        in_specs=[pl.BlockSpec((LKB1, HF), lambda i: (i, 0)),
                  pl.BlockSpec((QK, HF), lambda i: (0, 0))],
        out_specs=pl.BlockSpec((LKB1, QK), lambda i: (i, 0)),
        out_shape=jax.ShapeDtypeStruct((LK, QK), jnp.float32),
    )(k2, WK)

    scores, rmax, den, thr = pl.pallas_call(
        _scores_body,
        grid=(LQ // LQB,),
        in_specs=[pl.BlockSpec((LQB, F), lambda i: (i, 0)),
                  pl.BlockSpec((QK, F), lambda i: (0, 0)),
                  pl.BlockSpec((LK, QK), lambda i: (0, 0))],
        out_specs=[pl.BlockSpec((LQB, LK // 128, 128), lambda i: (i, 0, 0)),
                   pl.BlockSpec((LQB,), lambda i: (i,)),
                   pl.BlockSpec((LQB,), lambda i: (i,)),
                   pl.BlockSpec((LQB,), lambda i: (i,))],
        out_shape=[jax.ShapeDtypeStruct((LQ, LK // 128, 128), jnp.float32),
                   jax.ShapeDtypeStruct((LQ,), jnp.float32),
                   jax.ShapeDtypeStruct((LQ,), jnp.float32),
                   jax.ShapeDtypeStruct((LQ,), jnp.float32)],
    )(q2, WQ, w_k)

    sc_topk = functools.partial(
        pl.kernel,
        out_type=[jax.ShapeDtypeStruct((LQ * TOPK,), jnp.float32),
                  jax.ShapeDtypeStruct((LQ * TOPK,), jnp.int32)],
        mesh=plsc.VectorSubcoreMesh(core_axis_name="c", subcore_axis_name="s"),
        compiler_params=pltpu.CompilerParams(needs_layout_passes=False),
        scratch_types=[
            pltpu.VMEM((2 * LK,), jnp.float32),
            pltpu.VMEM((LK // 16 + 16,), jnp.int32),
            pltpu.VMEM((NBINS,), jnp.int32),
            pltpu.VMEM((CAP + 16,), jnp.float32),
            pltpu.VMEM((CAP + 16,), jnp.int32),
            pltpu.VMEM((RPW,), jnp.float32),
            pltpu.VMEM((RPW,), jnp.float32),
            pltpu.VMEM((RPW,), jnp.float32),
            pltpu.VMEM((RPW * TOPK,), jnp.float32),
            pltpu.VMEM((RPW * TOPK,), jnp.int32),
            pltpu.SemaphoreType.DMA,
        ],
    )(_sc_topk_body)

    p_flat, gi_flat = sc_topk(scores.reshape(LQ * LK), rmax, den, thr)

    w_v = pl.pallas_call(
        _wv_body,
        grid=(LK // LKB2,),
        in_specs=[pl.BlockSpec((LKB2, HF), lambda i: (i, 0)),
                  pl.BlockSpec((F, HF), lambda i: (0, 0))],
        out_specs=pl.BlockSpec((LKB2, F // 128, 128), lambda i: (i, 0, 0)),
        out_shape=jax.ShapeDtypeStruct((LK, F // 128, 128), jnp.float32),
    )(v2, WV)

    sc_gather = functools.partial(
        pl.kernel,
        out_type=jax.ShapeDtypeStruct((LQ * TOPK,), jnp.float32),
        mesh=plsc.VectorSubcoreMesh(core_axis_name="c", subcore_axis_name="s"),
        compiler_params=pltpu.CompilerParams(needs_layout_passes=False),
        scratch_types=[
            pltpu.VMEM((RPW * TOPK,), jnp.int32),
            pltpu.VMEM((RPW * TOPK,), jnp.float32),
            pltpu.SemaphoreType.DMA,
        ],
    )(_sc_gather_body)

    s_flat = sc_gather(gi_flat, w_v.reshape(LK * F))
    P = p_flat.reshape(LQ, TOPK)
    S = s_flat.reshape(LQ, TOPK)

    out = pl.pallas_call(
        _final_body,
        grid=(LQ // LQB,),
        in_specs=[pl.BlockSpec((LQB, TOPK), lambda i: (i, 0)),
                  pl.BlockSpec((LQ, TOPK), lambda i: (0, 0)),
                  pl.BlockSpec((LQB,), lambda i: (i,)),
                  pl.BlockSpec((LQB, F), lambda i: (i, 0))],
        out_specs=pl.BlockSpec((LQB, F), lambda i: (i, 0)),
        out_shape=jax.ShapeDtypeStruct((LQ, F), jnp.float32),
    )(P, S, rmax, q2)

    return out[None]


# speedup vs baseline: 18.2676x; 1.0167x over previous
import functools

import jax
import jax.numpy as jnp
from jax import lax
from jax.experimental import pallas as pl
from jax.experimental.pallas import tpu as pltpu
from jax.experimental.pallas import tpu_sc as plsc

F = 2048
HF = 1024
QK = 64
LQ = 2048
LK = 8192
TOPK = 32
GATE = 0.1
TEMP = QK ** -0.5
EPS = 1e-12

NW = 32
RPW = LQ // NW
NBINS = 512
CAP = 1024


def _l2n(x):
    n = jnp.sqrt(jnp.sum(x * x, axis=1, keepdims=True))
    return x / jnp.maximum(n, EPS)


def _scores_body(kin_ref, wkw_ref, q_ref, wq_ref, s_ref, m_ref, d_ref, t_ref,
                 wk_s):
    i = pl.program_id(0)

    @pl.when(i < LK // LKB1)
    def _wk_phase():
        y = lax.dot_general(kin_ref[...], wkw_ref[...],
                            (((1,), (1,)), ((), ())),
                            preferred_element_type=jnp.float32)
        wk_s[pl.ds(pl.multiple_of(i * LKB1, LKB1), LKB1), :] = _l2n(y)

    @pl.when(i >= LK // LKB1)
    def _score_phase():
        wq = lax.dot_general(q_ref[...], wq_ref[...], (((1,), (1,)), ((), ())),
                             preferred_element_type=jnp.float32)
        wq = _l2n(wq)
        s = lax.dot_general(wq, wk_s[...], (((1,), (1,)), ((), ())),
                            preferred_element_type=jnp.float32)
        s_ref[...] = s.reshape(LQB, LK // 128, 128)
        m = jnp.max(s, axis=1)
        m_ref[...] = m
        x = (s - m[:, None]) * TEMP
        ex = 1.0 + x * (1.0 + x * (0.5 + x * (1.0 / 6.0 + x * (1.0 / 24.0))))
        d_ref[...] = jnp.sum(ex, axis=1)
        mu = jnp.mean(s, axis=1)
        sg = jnp.sqrt(jnp.maximum(jnp.mean(s * s, axis=1) - mu * mu, 0.0))
        t_ref[...] = jnp.minimum(mu + 2.5 * sg, m)


def _wv_body(v_ref, wv_ref, o_ref):
    y = lax.dot_general(v_ref[...], wv_ref[...], (((1,), (1,)), ((), ())),
                        preferred_element_type=jnp.float32)
    o_ref[...] = _l2n(y).reshape(LKB2, F // 128, 128)


def _final_body(p_ref, s_ref, m_ref, q_ref, o_ref):
    g = lax.dot_general(p_ref[...], s_ref[...], (((1,), (1,)), ((), ())),
                        preferred_element_type=jnp.float32)
    msk = (m_ref[...] > GATE).astype(jnp.float32)
    o_ref[...] = g * msk[:, None] + q_ref[...]


def _merge_keep_top(ak, ai, bk, bi):
    rbk = lax.rev(bk, (0,))
    rbi = lax.rev(bi, (0,))
    m = ak >= rbk
    hk = jnp.where(m, ak, rbk)
    hi = jnp.where(m, ai, rbi)
    return plsc.sort_key_val(hk, hi)


def _merge32(ak, ai, bk, bi):
    rbk = lax.rev(bk, (0,))
    rbi = lax.rev(bi, (0,))
    m = ak <= rbk
    lk_ = jnp.where(m, ak, rbk)
    li_ = jnp.where(m, ai, rbi)
    hk_ = jnp.where(m, rbk, ak)
    hi_ = jnp.where(m, rbi, ai)
    lk2, li2 = plsc.sort_key_val(lk_, li_)
    hk2, hi2 = plsc.sort_key_val(hk_, hi_)
    return lk2, li2, hk2, hi2


def _sc_topk_body(scores_hbm, rmax_hbm, den_hbm, thr_hbm, p_hbm, gi_hbm,
                  rowbuf, flag, hist, cval, cidx, rm_loc, dn_loc,
                  th_loc, p_loc, gi_loc, rsem):
    wid = lax.axis_index("s") * 2 + lax.axis_index("c")
    base = wid * RPW
    pltpu.sync_copy(rmax_hbm.at[pl.ds(base, RPW)], rm_loc)
    pltpu.sync_copy(den_hbm.at[pl.ds(base, RPW)], dn_loc)
    pltpu.sync_copy(thr_hbm.at[pl.ds(base, RPW)], th_loc)
    iota16 = lax.iota(jnp.int32, 16)
    ones16 = jnp.ones((16,), jnp.int32)

    pltpu.async_copy(scores_hbm.at[pl.ds(base * LK, LK)],
                     rowbuf.at[pl.ds(0, LK)], rsem)

    def row_step(r, carry_unused):
        pbase = (r % 2) * LK
        buf = rowbuf.at[pl.ds(pbase, LK)]
        pltpu.make_async_copy(scores_hbm.at[pl.ds((base + r) * LK, LK)],
                              buf, rsem).wait()

        @pl.when(r + 1 < RPW)
        def _prefetch():
            pltpu.async_copy(scores_hbm.at[pl.ds((base + r + 1) * LK, LK)],
                             rowbuf.at[pl.ds(((r + 1) % 2) * LK, LK)], rsem)

        rsel = jnp.broadcast_to(r, (16,)).astype(jnp.int32)
        t0v = plsc.load_gather(th_loc, [rsel])

        def ab_step(g, ptrf):
            acc = jnp.zeros((16,), jnp.float32)
            for u in range(16):
                v = buf[pl.ds((g * 16 + u) * 16, 16)]
                acc = jnp.where(iota16 == u, jnp.max(v), acc)
            m = acc >= t0v
            nf = plsc.all_reduce_population_count(m)[0]
            plsc.store_compressed(flag.at[pl.ds(ptrf, 16)], g * 16 + iota16,
                                  mask=m)
            return ptrf + nf

        ptrf = lax.fori_loop(0, LK // 256, ab_step, jnp.int32(0))

        def c_step(g, carry):
            ptr, tot = carry
            fv = flag[pl.ds(g * 16, 16)]
            for u in range(16):
                active = (g * 16 + u) < ptrf
                c = jnp.where(active, fv[u], 0)
                v = buf[pl.ds(c * 16, 16)]
                m = (v >= t0v) & active
                cnt = plsc.all_reduce_population_count(m)[0]
                plsc.store_compressed(cval.at[pl.ds(ptr, 16)], v, mask=m)
                plsc.store_compressed(cidx.at[pl.ds(ptr, 16)],
                                      c * 16 + iota16, mask=m)
                ptr = jnp.minimum(ptr + cnt, CAP - 16)
                tot = tot + cnt
            return (ptr, tot)

        ptr, tot = lax.fori_loop(0, (ptrf + 15) // 16, c_step,
                                 (jnp.int32(0), jnp.int32(0)))

        def _fallback(_):
            for i in range(NBINS // 16):
                hist[pl.ds(i * 16, 16)] = jnp.zeros((16,), jnp.int32)

            def h_step(i, _):
                v = buf[pl.ds(i * 16, 16)]
                b = jnp.clip(((v + 1.0) * (NBINS / 2.0)).astype(jnp.int32),
                             0, NBINS - 1)
                plsc.addupdate_scatter(hist, [b], ones16)
                return 0

            lax.fori_loop(0, LK // 16, h_step, 0)

            def t_step(i, carry):
                cum, bfound, bbin = carry
                blk_id = (NBINS // 16 - 1) - i
                blk = hist[pl.ds(blk_id * 16, 16)]
                rcum = plsc.cumsum(lax.rev(blk, (0,))) + cum
                m = rcum >= TOPK
                any_m = jnp.max(m.astype(jnp.int32))
                ffs = jnp.max(plsc.all_reduce_ffs(m))
                cand_bin = blk_id * 16 + 15 - ffs
                bbin = jnp.where(bfound == 0,
                                 jnp.where(any_m == 1, cand_bin, bbin), bbin)
                bfound = jnp.maximum(bfound, any_m)
                return (cum + jnp.sum(blk), bfound, bbin)

            _, _, bbin = lax.fori_loop(
                0, NBINS // 16, t_step,
                (jnp.int32(0), jnp.int32(0), jnp.int32(0)))
            thresh = bbin.astype(jnp.float32) * (2.0 / NBINS) - 1.0

            def c2_step(i, p2):
                v = buf[pl.ds(i * 16, 16)]
                m = v >= thresh
                cnt = plsc.all_reduce_population_count(m)[0]
                plsc.store_compressed(cval.at[pl.ds(p2, 16)], v, mask=m)
                plsc.store_compressed(cidx.at[pl.ds(p2, 16)], i * 16 + iota16,
                                      mask=m)
                return jnp.minimum(p2 + cnt, CAP - 16)

            return lax.fori_loop(0, LK // 16, c2_step, jnp.int32(0))

        ptr = lax.cond((tot < TOPK) | (tot > CAP - 16), _fallback,
                       lambda _: ptr, 0)
        cval[pl.ds(ptr, 16)] = jnp.full((16,), -2.0, jnp.float32)
        cidx[pl.ds(ptr, 16)] = jnp.zeros((16,), jnp.int32)

        def s_step(t, st):
            lok, loi, hik, hii = st
            ck = cval[pl.ds(t * 16, 16)]
            ci = cidx[pl.ds(t * 16, 16)]
            nk, ni = plsc.sort_key_val(ck, ci)
            h1k, h1i = _merge_keep_top(lok, loi, nk, ni)
            return _merge32(h1k, h1i, hik, hii)

        init = (jnp.full((16,), -3.0, jnp.float32), jnp.zeros((16,), jnp.int32),
                jnp.full((16,), -3.0, jnp.float32), jnp.zeros((16,), jnp.int32))
        lok, loi, hik, hii = lax.fori_loop(0, (ptr + 15) // 16, s_step, init)

        kd0 = lax.rev(hik, (0,))
        kd1 = lax.rev(lok, (0,))
        id0 = lax.rev(hii, (0,))
        id1 = lax.rev(loi, (0,))
        rmv = plsc.load_gather(rm_loc, [jnp.broadcast_to(r, (16,)).astype(jnp.int32)])
        dnv = plsc.load_gather(dn_loc, [jnp.broadcast_to(r, (16,)).astype(jnp.int32)])
        p_loc[pl.ds(r * TOPK, 16)] = jnp.exp((kd0 - rmv) * TEMP) / dnv
        p_loc[pl.ds(r * TOPK + 16, 16)] = jnp.exp((kd1 - rmv) * TEMP) / dnv
        col = base + r
        gi_loc[pl.ds(r * TOPK, 16)] = id0 * F + col
        gi_loc[pl.ds(r * TOPK + 16, 16)] = id1 * F + col
        return 0

    lax.fori_loop(0, RPW, row_step, 0)

    pltpu.sync_copy(p_loc, p_hbm.at[pl.ds(base * TOPK, RPW * TOPK)])
    pltpu.sync_copy(gi_loc, gi_hbm.at[pl.ds(base * TOPK, RPW * TOPK)])


def _sc_gather_body(gi_hbm, wv_hbm, s_hbm, gi_loc, s_loc, gsem):
    wid = lax.axis_index("s") * 2 + lax.axis_index("c")
    base = wid * RPW
    pltpu.sync_copy(gi_hbm.at[pl.ds(base * TOPK, RPW * TOPK)], gi_loc)
    handles = []
    for c in range(RPW * TOPK // 128):
        handles.append(pltpu.async_copy(
            wv_hbm.at[gi_loc.at[pl.ds(c * 128, 128)]],
            s_loc.at[pl.ds(c * 128, 128)], gsem))
    for h in handles:
        h.wait()
    pltpu.sync_copy(s_loc, s_hbm.at[pl.ds(base * TOPK, RPW * TOPK)])


LQB = 256
LKB1 = 1024
LKB2 = 512


@jax.jit
def kernel(query, key_in, value, WQ, WK, WV):
    q2 = query[0]
    k2 = key_in[0]
    v2 = value[0]

    nk = LK // LKB1
    scores, rmax, den, thr = pl.pallas_call(
        _scores_body,
        grid=(nk + LQ // LQB,),
        in_specs=[
            pl.BlockSpec((LKB1, HF), lambda i: (jnp.minimum(i, nk - 1), 0)),
            pl.BlockSpec((QK, HF), lambda i: (0, 0)),
            pl.BlockSpec((LQB, F), lambda i: (jnp.maximum(i - nk, 0), 0)),
            pl.BlockSpec((QK, F), lambda i: (0, 0)),
        ],
        out_specs=[pl.BlockSpec((LQB, LK // 128, 128),
                                lambda i: (jnp.maximum(i - nk, 0), 0, 0)),
                   pl.BlockSpec((LQB,), lambda i: (jnp.maximum(i - nk, 0),)),
                   pl.BlockSpec((LQB,), lambda i: (jnp.maximum(i - nk, 0),)),
                   pl.BlockSpec((LQB,), lambda i: (jnp.maximum(i - nk, 0),))],
        out_shape=[jax.ShapeDtypeStruct((LQ, LK // 128, 128), jnp.float32),
                   jax.ShapeDtypeStruct((LQ,), jnp.float32),
                   jax.ShapeDtypeStruct((LQ,), jnp.float32),
                   jax.ShapeDtypeStruct((LQ,), jnp.float32)],
        scratch_shapes=[pltpu.VMEM((LK, QK), jnp.float32)],
    )(k2, WK, q2, WQ)

    sc_topk = functools.partial(
        pl.kernel,
        out_type=[jax.ShapeDtypeStruct((LQ * TOPK,), jnp.float32),
                  jax.ShapeDtypeStruct((LQ * TOPK,), jnp.int32)],
        mesh=plsc.VectorSubcoreMesh(core_axis_name="c", subcore_axis_name="s"),
        compiler_params=pltpu.CompilerParams(needs_layout_passes=False),
        scratch_types=[
            pltpu.VMEM((2 * LK,), jnp.float32),
            pltpu.VMEM((LK // 16 + 16,), jnp.int32),
            pltpu.VMEM((NBINS,), jnp.int32),
            pltpu.VMEM((CAP + 16,), jnp.float32),
            pltpu.VMEM((CAP + 16,), jnp.int32),
            pltpu.VMEM((RPW,), jnp.float32),
            pltpu.VMEM((RPW,), jnp.float32),
            pltpu.VMEM((RPW,), jnp.float32),
            pltpu.VMEM((RPW * TOPK,), jnp.float32),
            pltpu.VMEM((RPW * TOPK,), jnp.int32),
            pltpu.SemaphoreType.DMA,
        ],
    )(_sc_topk_body)

    p_flat, gi_flat = sc_topk(scores.reshape(LQ * LK), rmax, den, thr)

    w_v = pl.pallas_call(
        _wv_body,
        grid=(LK // LKB2,),
        in_specs=[pl.BlockSpec((LKB2, HF), lambda i: (i, 0)),
                  pl.BlockSpec((F, HF), lambda i: (0, 0))],
        out_specs=pl.BlockSpec((LKB2, F // 128, 128), lambda i: (i, 0, 0)),
        out_shape=jax.ShapeDtypeStruct((LK, F // 128, 128), jnp.float32),
    )(v2, WV)

    sc_gather = functools.partial(
        pl.kernel,
        out_type=jax.ShapeDtypeStruct((LQ * TOPK,), jnp.float32),
        mesh=plsc.VectorSubcoreMesh(core_axis_name="c", subcore_axis_name="s"),
        compiler_params=pltpu.CompilerParams(needs_layout_passes=False),
        scratch_types=[
            pltpu.VMEM((RPW * TOPK,), jnp.int32),
            pltpu.VMEM((RPW * TOPK,), jnp.float32),
            pltpu.SemaphoreType.DMA,
        ],
    )(_sc_gather_body)

    s_flat = sc_gather(gi_flat, w_v.reshape(LK * F))
    P = p_flat.reshape(LQ, TOPK)
    S = s_flat.reshape(LQ, TOPK)

    out = pl.pallas_call(
        _final_body,
        grid=(LQ // LQB,),
        in_specs=[pl.BlockSpec((LQB, TOPK), lambda i: (i, 0)),
                  pl.BlockSpec((LQ, TOPK), lambda i: (0, 0)),
                  pl.BlockSpec((LQB,), lambda i: (i,)),
                  pl.BlockSpec((LQB, F), lambda i: (i, 0))],
        out_specs=pl.BlockSpec((LQB, F), lambda i: (i, 0)),
        out_shape=jax.ShapeDtypeStruct((LQ, F), jnp.float32),
    )(P, S, rmax, q2)

    return out[None]
